# Initial kernel scaffold; baseline (speedup 1.0000x reference)
import jax
import jax.numpy as jnp
from jax.experimental import pallas as pl


def kernel(x, edge_index, edge_weight, batch, W1, b1, W2, b2, Wl, bl):
    raise NotImplementedError("write your pallas kernel here")


# speedup vs baseline: 9.9165x; 9.9165x over previous
import functools

import jax
import jax.numpy as jnp
from jax import lax
from jax.experimental import pallas as pl
from jax.experimental.pallas import tpu as pltpu
from jax.experimental.pallas import tpu_sc as plsc

N = 10000
D = 128
H = 64
C = 2
G = 64
E = 320000

NC = 2
NS = 16
NW = NC * NS

NP = 10240
RPT = NP // NS
CH = 128
EF = E + N
EP = ((EF + NW * CH - 1) // (NW * CH)) * (NW * CH)
ET = EP // NW
NCHUNK = ET // CH

_MESH = dict(core_axis_name="c", subcore_axis_name="s",
             num_cores=NC, num_subcores=NS)


@functools.partial(
    pl.kernel,
    out_type=jax.ShapeDtypeStruct((NC, NP), jnp.float32),
    mesh=plsc.VectorSubcoreMesh(**_MESH),
    scratch_types=[
        pltpu.VMEM_SHARED((NP,), jnp.float32),
        pltpu.VMEM((CH,), jnp.int32),
        pltpu.VMEM((CH,), jnp.float32),
        pltpu.VMEM((RPT,), jnp.float32),
    ],
)
def _deg_kernel(dst_hbm, ew_hbm, out_hbm, deg_sh, dstv, ewv, buf):
    cid = lax.axis_index("c")
    sid = lax.axis_index("s")
    wid = sid * NC + cid

    def zb(i, _):
        buf[pl.ds(i * 16, 16)] = jnp.zeros((16,), jnp.float32)
        return 0
    lax.fori_loop(0, RPT // 16, zb, 0)
    pltpu.sync_copy(buf, deg_sh.at[pl.ds(sid * RPT, RPT)])
    plsc.subcore_barrier()

    def body(i, _):
        base = wid * ET + i * CH
        pltpu.sync_copy(dst_hbm.at[pl.ds(base, CH)], dstv)
        pltpu.sync_copy(ew_hbm.at[pl.ds(base, CH)], ewv)
        pltpu.sync_copy(ewv, deg_sh.at[dstv], add=True)
        return 0
    lax.fori_loop(0, NCHUNK, body, 0)
    plsc.subcore_barrier()

    pltpu.sync_copy(deg_sh.at[pl.ds(sid * RPT, RPT)], buf)
    pltpu.sync_copy(buf, out_hbm.at[cid, pl.ds(sid * RPT, RPT)])


@functools.partial(
    pl.kernel,
    out_type=jax.ShapeDtypeStruct((NC, NP, H), jnp.float32),
    mesh=plsc.VectorSubcoreMesh(**_MESH),
    scratch_types=[
        pltpu.VMEM_SHARED((NP, H), jnp.float32),
        pltpu.VMEM((CH,), jnp.int32),
        pltpu.VMEM((CH,), jnp.int32),
        pltpu.VMEM((CH,), jnp.float32),
        pltpu.VMEM((CH, H), jnp.float32),
        pltpu.SemaphoreType.DMA,
    ],
    compiler_params=pltpu.CompilerParams(use_tc_tiling_on_sc=False),
)
def _agg_kernel(tab_hbm, src_hbm, dst_hbm, ew_hbm, out_hbm,
                acc_sh, srcv, dstv, ewv, rows, sem):
    cid = lax.axis_index("c")
    sid = lax.axis_index("s")
    wid = sid * NC + cid

    def zb(i, _):
        rows[i // (H // 16), pl.ds((i % (H // 16)) * 16, 16)] = (
            jnp.zeros((16,), jnp.float32))
        return 0
    lax.fori_loop(0, CH * (H // 16), zb, 0)

    def zc(j, _):
        pltpu.sync_copy(rows, acc_sh.at[pl.ds(sid * RPT + j * CH, CH)])
        return 0
    lax.fori_loop(0, RPT // CH, zc, 0)
    plsc.subcore_barrier()

    def body(i, _):
        base = wid * ET + i * CH
        pltpu.sync_copy(src_hbm.at[pl.ds(base, CH)], srcv)
        pltpu.sync_copy(dst_hbm.at[pl.ds(base, CH)], dstv)
        pltpu.sync_copy(ew_hbm.at[pl.ds(base, CH)], ewv)
        pltpu.async_copy(tab_hbm.at[srcv], rows, sem).wait()

        def scale(g, _):
            ew16 = ewv[pl.ds(g * 16, 16)]
            for j in range(16):
                e = g * 16 + j
                s = ew16[j]
                for f in range(H // 16):
                    rows[e, pl.ds(f * 16, 16)] = (
                        rows[e, pl.ds(f * 16, 16)] * s)
            return 0
        lax.fori_loop(0, CH // 16, scale, 0)

        pltpu.sync_copy(rows, acc_sh.at[dstv], add=True)
        return 0
    lax.fori_loop(0, NCHUNK, body, 0)
    plsc.subcore_barrier()

    def dump(j, _):
        pltpu.sync_copy(acc_sh.at[pl.ds(sid * RPT + j * CH, CH)], rows)
        pltpu.sync_copy(rows, out_hbm.at[cid, pl.ds(sid * RPT + j * CH, CH)])
        return 0
    lax.fori_loop(0, RPT // CH, dump, 0)


BR = 1024
GRID = NP // BR


def _dinv(dga, dgb):
    deg = dga + dgb
    return jnp.where(deg > 0, lax.rsqrt(deg), 0.0)


def _mm1_body(x_ref, w_ref, dga_ref, dgb_ref, o_ref):
    dinv = _dinv(dga_ref[...], dgb_ref[...])
    o_ref[...] = jnp.dot(x_ref[...], w_ref[...],
                         preferred_element_type=jnp.float32) * dinv


_mm1 = pl.pallas_call(
    _mm1_body,
    grid=(GRID,),
    in_specs=[
        pl.BlockSpec((BR, D), lambda i: (i, 0)),
        pl.BlockSpec((D, H), lambda i: (0, 0)),
        pl.BlockSpec((BR, 1), lambda i: (i, 0)),
        pl.BlockSpec((BR, 1), lambda i: (i, 0)),
    ],
    out_specs=pl.BlockSpec((BR, H), lambda i: (i, 0)),
    out_shape=jax.ShapeDtypeStruct((NP, H), jnp.float32),
)


def _mm2_body(a_ref, b_ref, dga_ref, dgb_ref, b1_ref, w_ref, o_ref):
    dinv = _dinv(dga_ref[...], dgb_ref[...])
    h = jnp.maximum((a_ref[...] + b_ref[...]) * dinv + b1_ref[...], 0.0)
    o_ref[...] = jnp.dot(h, w_ref[...],
                         preferred_element_type=jnp.float32) * dinv


_mm2 = pl.pallas_call(
    _mm2_body,
    grid=(GRID,),
    in_specs=[
        pl.BlockSpec((BR, H), lambda i: (i, 0)),
        pl.BlockSpec((BR, H), lambda i: (i, 0)),
        pl.BlockSpec((BR, 1), lambda i: (i, 0)),
        pl.BlockSpec((BR, 1), lambda i: (i, 0)),
        pl.BlockSpec((1, H), lambda i: (0, 0)),
        pl.BlockSpec((H, H), lambda i: (0, 0)),
    ],
    out_specs=pl.BlockSpec((BR, H), lambda i: (i, 0)),
    out_shape=jax.ShapeDtypeStruct((NP, H), jnp.float32),
)


def _pool_body(a_ref, b_ref, dga_ref, dgb_ref, b2_ref, bat_ref, wl_ref,
               bl_ref, out_ref, pooled_ref, sums, cnts):
    i = pl.program_id(0)

    @pl.when(i == 0)
    def _():
        sums[...] = jnp.zeros_like(sums)
        cnts[...] = jnp.zeros_like(cnts)

    dinv = _dinv(dga_ref[...], dgb_ref[...])
    h2 = jnp.maximum((a_ref[...] + b_ref[...]) * dinv + b2_ref[...], 0.0)
    sel = (bat_ref[...] == lax.broadcasted_iota(jnp.int32, (BR, G), 1)
           ).astype(jnp.float32)
    sums[...] += lax.dot_general(sel, h2, (((0,), (0,)), ((), ())),
                                 preferred_element_type=jnp.float32)
    cnts[...] += lax.dot_general(sel, jnp.ones((BR, 1), jnp.float32),
                                 (((0,), (0,)), ((), ())),
                                 preferred_element_type=jnp.float32)

    @pl.when(i == GRID - 1)
    def _():
        pooled = sums[...] / jnp.maximum(cnts[...], 1.0)
        pooled_ref[...] = pooled
        out_ref[...] = jnp.dot(pooled, wl_ref[...],
                               preferred_element_type=jnp.float32) + bl_ref[...]


_pool = pl.pallas_call(
    _pool_body,
    grid=(GRID,),
    in_specs=[
        pl.BlockSpec((BR, H), lambda i: (i, 0)),
        pl.BlockSpec((BR, H), lambda i: (i, 0)),
        pl.BlockSpec((BR, 1), lambda i: (i, 0)),
        pl.BlockSpec((BR, 1), lambda i: (i, 0)),
        pl.BlockSpec((1, H), lambda i: (0, 0)),
        pl.BlockSpec((BR, 1), lambda i: (i, 0)),
        pl.BlockSpec((H, C), lambda i: (0, 0)),
        pl.BlockSpec((1, C), lambda i: (0, 0)),
    ],
    out_specs=[
        pl.BlockSpec((G, C), lambda i: (0, 0)),
        pl.BlockSpec((G, H), lambda i: (0, 0)),
    ],
    out_shape=[
        jax.ShapeDtypeStruct((G, C), jnp.float32),
        jax.ShapeDtypeStruct((G, H), jnp.float32),
    ],
    scratch_shapes=[
        pltpu.VMEM((G, H), jnp.float32),
        pltpu.VMEM((G, 1), jnp.float32),
    ],
)


def kernel(x, edge_index, edge_weight, batch, W1, b1, W2, b2, Wl, bl):
    src, dst = edge_index[0], edge_index[1]
    loop_idx = jnp.arange(N, dtype=src.dtype)
    srcf = jnp.concatenate([src, loop_idx])
    dstf = jnp.concatenate([dst, loop_idx])
    ewf = jnp.concatenate([edge_weight, jnp.ones((N,), edge_weight.dtype)])

    pad = EP - EF
    srcf = jnp.pad(srcf, (0, pad))
    dstf = jnp.pad(dstf, (0, pad))
    ewf = jnp.pad(ewf, (0, pad))
    xp = jnp.pad(x, ((0, NP - N), (0, 0)))
    batp = jnp.pad(batch, (0, NP - N), constant_values=G).reshape(NP, 1)

    deg2 = _deg_kernel(dstf, ewf)
    dga = deg2[0].reshape(NP, 1)
    dgb = deg2[1].reshape(NP, 1)

    xs1 = _mm1(xp, W1, dga, dgb)
    acc1 = _agg_kernel(xs1, srcf, dstf, ewf)
    xs2 = _mm2(acc1[0], acc1[1], dga, dgb, b1.reshape(1, H), W2)
    acc2 = _agg_kernel(xs2, srcf, dstf, ewf)
    out, pooled = _pool(acc2[0], acc2[1], dga, dgb, b2.reshape(1, H),
                        batp, Wl, bl.reshape(1, C))
    return (out, pooled)


# speedup vs baseline: 14.8428x; 1.4968x over previous
import functools

import jax
import jax.numpy as jnp
from jax import lax
from jax.experimental import pallas as pl
from jax.experimental.pallas import tpu as pltpu
from jax.experimental.pallas import tpu_sc as plsc

N = 10000
D = 128
H = 64
C = 2
G = 64
E = 320000

NC = 2
NS = 16
NW = NC * NS

NP = 10240
RPT = NP // NS
CH = 128
EF = E + N
_NC0 = (EF + NW * CH - 1) // (NW * CH)
NCHUNK = _NC0 + (_NC0 % 2)
ET = NCHUNK * CH
EP = NW * ET

_MESH = dict(core_axis_name="c", subcore_axis_name="s",
             num_cores=NC, num_subcores=NS)


@functools.partial(
    pl.kernel,
    out_type=jax.ShapeDtypeStruct((NC, NP), jnp.float32),
    mesh=plsc.VectorSubcoreMesh(**_MESH),
    scratch_types=[
        pltpu.VMEM_SHARED((NP,), jnp.float32),
        pltpu.VMEM((NCHUNK, CH), jnp.int32),
        pltpu.VMEM((NCHUNK, CH), jnp.float32),
        pltpu.VMEM((RPT,), jnp.float32),
        pltpu.SemaphoreType.DMA,
    ],
    compiler_params=pltpu.CompilerParams(use_tc_tiling_on_sc=False),
)
def _deg_kernel(dst_hbm, ew_hbm, out_hbm, deg_sh, dstv, ewv, buf, sem):
    cid = lax.axis_index("c")
    sid = lax.axis_index("s")
    wid = sid * NC + cid

    def zb(i, _):
        buf[pl.ds(i * 16, 16)] = jnp.zeros((16,), jnp.float32)
        return 0
    lax.fori_loop(0, RPT // 16, zb, 0)
    pltpu.sync_copy(buf, deg_sh.at[pl.ds(sid * RPT, RPT)])
    plsc.subcore_barrier()

    pltpu.sync_copy(dst_hbm.at[pl.ds(wid * NCHUNK, NCHUNK)], dstv)
    pltpu.sync_copy(ew_hbm.at[pl.ds(wid * NCHUNK, NCHUNK)], ewv)

    def fire(i, _):
        pltpu.sync_copy(ewv.at[i], deg_sh.at[dstv.at[i]], add=True)
        return 0
    lax.fori_loop(0, NCHUNK, fire, 0)
    plsc.subcore_barrier()

    pltpu.sync_copy(deg_sh.at[pl.ds(sid * RPT, RPT)], buf)
    pltpu.sync_copy(buf, out_hbm.at[cid, pl.ds(sid * RPT, RPT)])


@functools.partial(
    pl.kernel,
    out_type=jax.ShapeDtypeStruct((NC, NP, H), jnp.float32),
    mesh=plsc.VectorSubcoreMesh(**_MESH),
    scratch_types=[
        pltpu.VMEM_SHARED((NP, H), jnp.float32),
        pltpu.VMEM((NCHUNK, CH), jnp.int32),
        pltpu.VMEM((NCHUNK, CH), jnp.int32),
        pltpu.VMEM((NCHUNK, CH), jnp.float32),
        pltpu.VMEM((CH, H), jnp.float32),
        pltpu.VMEM((CH, H), jnp.float32),
        pltpu.SemaphoreType.DMA,
        pltpu.SemaphoreType.DMA,
        pltpu.SemaphoreType.DMA,
        pltpu.SemaphoreType.DMA,
    ],
    compiler_params=pltpu.CompilerParams(use_tc_tiling_on_sc=False),
)
def _agg_kernel(tab_hbm, src_hbm, dst_hbm, ew_hbm, out_hbm,
                acc_sh, srcv, dstv, ewv, r0, r1, g0, g1, s0, s1):
    cid = lax.axis_index("c")
    sid = lax.axis_index("s")
    wid = sid * NC + cid
    rows = (r0, r1)
    gsem = (g0, g1)
    ssem = (s0, s1)

    def zb(i, _):
        r0[i // (H // 16), pl.ds((i % (H // 16)) * 16, 16)] = (
            jnp.zeros((16,), jnp.float32))
        return 0
    lax.fori_loop(0, CH * (H // 16), zb, 0)

    def zc(j, _):
        pltpu.sync_copy(r0, acc_sh.at[pl.ds(sid * RPT + j * CH, CH)])
        return 0
    lax.fori_loop(0, RPT // CH, zc, 0)
    plsc.subcore_barrier()

    pltpu.sync_copy(src_hbm.at[pl.ds(wid * NCHUNK, NCHUNK)], srcv)
    pltpu.sync_copy(dst_hbm.at[pl.ds(wid * NCHUNK, NCHUNK)], dstv)
    pltpu.sync_copy(ew_hbm.at[pl.ds(wid * NCHUNK, NCHUNK)], ewv)

    pltpu.async_copy(tab_hbm.at[srcv.at[0]], r0, g0)

    def outer(h, _):
        for b in range(2):
            i = h * 2 + b
            rb, gb = rows[b], gsem[b]
            ro, go = rows[1 - b], gsem[1 - b]
            pltpu.make_async_copy(tab_hbm.at[srcv.at[i]], rb, gb).wait()

            @pl.when(i + 1 < NCHUNK)
            def _():
                pltpu.async_copy(tab_hbm.at[srcv.at[i + 1]], ro, go)

            def scale(g, _):
                ew16 = ewv[i, pl.ds(g * 16, 16)]
                for j in range(16):
                    e = g * 16 + j
                    s = ew16[j]
                    for f in range(H // 16):
                        rb[e, pl.ds(f * 16, 16)] = (
                            rb[e, pl.ds(f * 16, 16)] * s)
                return 0
            lax.fori_loop(0, CH // 16, scale, 0)

            pltpu.sync_copy(rb, acc_sh.at[dstv.at[i]], add=True)
        return 0
    lax.fori_loop(0, NCHUNK // 2, outer, 0)
    plsc.subcore_barrier()

    def dump(j, _):
        pltpu.sync_copy(acc_sh.at[pl.ds(sid * RPT + j * CH, CH)], r0)
        pltpu.sync_copy(r0, out_hbm.at[cid, pl.ds(sid * RPT + j * CH, CH)])
        return 0
    lax.fori_loop(0, RPT // CH, dump, 0)


BR = 1024
GRID = NP // BR


def _dinv(dga, dgb):
    deg = dga + dgb
    return jnp.where(deg > 0, lax.rsqrt(deg), 0.0)


def _mm1_body(x_ref, w_ref, dga_ref, dgb_ref, o_ref):
    dinv = _dinv(dga_ref[...], dgb_ref[...])
    o_ref[...] = jnp.dot(x_ref[...], w_ref[...],
                         preferred_element_type=jnp.float32) * dinv


_mm1 = pl.pallas_call(
    _mm1_body,
    grid=(GRID,),
    in_specs=[
        pl.BlockSpec((BR, D), lambda i: (i, 0)),
        pl.BlockSpec((D, H), lambda i: (0, 0)),
        pl.BlockSpec((BR, 1), lambda i: (i, 0)),
        pl.BlockSpec((BR, 1), lambda i: (i, 0)),
    ],
    out_specs=pl.BlockSpec((BR, H), lambda i: (i, 0)),
    out_shape=jax.ShapeDtypeStruct((NP, H), jnp.float32),
)


def _mm2_body(a_ref, b_ref, dga_ref, dgb_ref, b1_ref, w_ref, o_ref):
    dinv = _dinv(dga_ref[...], dgb_ref[...])
    h = jnp.maximum((a_ref[...] + b_ref[...]) * dinv + b1_ref[...], 0.0)
    o_ref[...] = jnp.dot(h, w_ref[...],
                         preferred_element_type=jnp.float32) * dinv


_mm2 = pl.pallas_call(
    _mm2_body,
    grid=(GRID,),
    in_specs=[
        pl.BlockSpec((BR, H), lambda i: (i, 0)),
        pl.BlockSpec((BR, H), lambda i: (i, 0)),
        pl.BlockSpec((BR, 1), lambda i: (i, 0)),
        pl.BlockSpec((BR, 1), lambda i: (i, 0)),
        pl.BlockSpec((1, H), lambda i: (0, 0)),
        pl.BlockSpec((H, H), lambda i: (0, 0)),
    ],
    out_specs=pl.BlockSpec((BR, H), lambda i: (i, 0)),
    out_shape=jax.ShapeDtypeStruct((NP, H), jnp.float32),
)


def _pool_body(a_ref, b_ref, dga_ref, dgb_ref, b2_ref, bat_ref, wl_ref,
               bl_ref, out_ref, pooled_ref, sums, cnts):
    i = pl.program_id(0)

    @pl.when(i == 0)
    def _():
        sums[...] = jnp.zeros_like(sums)
        cnts[...] = jnp.zeros_like(cnts)

    dinv = _dinv(dga_ref[...], dgb_ref[...])
    h2 = jnp.maximum((a_ref[...] + b_ref[...]) * dinv + b2_ref[...], 0.0)
    sel = (bat_ref[...] == lax.broadcasted_iota(jnp.int32, (BR, G), 1)
           ).astype(jnp.float32)
    sums[...] += lax.dot_general(sel, h2, (((0,), (0,)), ((), ())),
                                 preferred_element_type=jnp.float32)
    cnts[...] += lax.dot_general(sel, jnp.ones((BR, 1), jnp.float32),
                                 (((0,), (0,)), ((), ())),
                                 preferred_element_type=jnp.float32)

    @pl.when(i == GRID - 1)
    def _():
        pooled = sums[...] / jnp.maximum(cnts[...], 1.0)
        pooled_ref[...] = pooled
        out_ref[...] = jnp.dot(pooled, wl_ref[...],
                               preferred_element_type=jnp.float32) + bl_ref[...]


_pool = pl.pallas_call(
    _pool_body,
    grid=(GRID,),
    in_specs=[
        pl.BlockSpec((BR, H), lambda i: (i, 0)),
        pl.BlockSpec((BR, H), lambda i: (i, 0)),
        pl.BlockSpec((BR, 1), lambda i: (i, 0)),
        pl.BlockSpec((BR, 1), lambda i: (i, 0)),
        pl.BlockSpec((1, H), lambda i: (0, 0)),
        pl.BlockSpec((BR, 1), lambda i: (i, 0)),
        pl.BlockSpec((H, C), lambda i: (0, 0)),
        pl.BlockSpec((1, C), lambda i: (0, 0)),
    ],
    out_specs=[
        pl.BlockSpec((G, C), lambda i: (0, 0)),
        pl.BlockSpec((G, H), lambda i: (0, 0)),
    ],
    out_shape=[
        jax.ShapeDtypeStruct((G, C), jnp.float32),
        jax.ShapeDtypeStruct((G, H), jnp.float32),
    ],
    scratch_shapes=[
        pltpu.VMEM((G, H), jnp.float32),
        pltpu.VMEM((G, 1), jnp.float32),
    ],
)


def kernel(x, edge_index, edge_weight, batch, W1, b1, W2, b2, Wl, bl):
    src, dst = edge_index[0], edge_index[1]
    loop_idx = jnp.arange(N, dtype=src.dtype)
    srcf = jnp.concatenate([src, loop_idx])
    dstf = jnp.concatenate([dst, loop_idx])
    ewf = jnp.concatenate([edge_weight, jnp.ones((N,), edge_weight.dtype)])

    pad = EP - EF
    srcf = jnp.pad(srcf, (0, pad)).reshape(EP // CH, CH)
    dstf = jnp.pad(dstf, (0, pad)).reshape(EP // CH, CH)
    ewf = jnp.pad(ewf, (0, pad)).reshape(EP // CH, CH)
    xp = jnp.pad(x, ((0, NP - N), (0, 0)))
    batp = jnp.pad(batch, (0, NP - N), constant_values=G).reshape(NP, 1)

    deg2 = _deg_kernel(dstf, ewf)
    dga = deg2[0].reshape(NP, 1)
    dgb = deg2[1].reshape(NP, 1)

    xs1 = _mm1(xp, W1, dga, dgb)
    acc1 = _agg_kernel(xs1, srcf, dstf, ewf)
    xs2 = _mm2(acc1[0], acc1[1], dga, dgb, b1.reshape(1, H), W2)
    acc2 = _agg_kernel(xs2, srcf, dstf, ewf)
    out, pooled = _pool(acc2[0], acc2[1], dga, dgb, b2.reshape(1, H),
                        batp, Wl, bl.reshape(1, C))
    return (out, pooled)


# speedup vs baseline: 18.0112x; 1.2135x over previous
import functools

import jax
import jax.numpy as jnp
from jax import lax
from jax.experimental import pallas as pl
from jax.experimental.pallas import tpu as pltpu
from jax.experimental.pallas import tpu_sc as plsc

N = 10000
D = 128
H = 64
C = 2
G = 64
E = 320000

NC = 2
NS = 16
NW = NC * NS

NP = 10240
RPT = NP // NS
CH = 128
EF = E + N
_NC0 = (EF + NW * CH - 1) // (NW * CH)
NCHUNK = ((_NC0 + 2) // 3) * 3
ET = NCHUNK * CH
EP = NW * ET

_MESH = dict(core_axis_name="c", subcore_axis_name="s",
             num_cores=NC, num_subcores=NS)


@functools.partial(
    pl.kernel,
    out_type=jax.ShapeDtypeStruct((NC, NP), jnp.float32),
    mesh=plsc.VectorSubcoreMesh(**_MESH),
    scratch_types=[
        pltpu.VMEM_SHARED((NP,), jnp.float32),
        pltpu.VMEM((NCHUNK, CH), jnp.int32),
        pltpu.VMEM((NCHUNK, CH), jnp.float32),
        pltpu.VMEM((RPT,), jnp.float32),
        pltpu.SemaphoreType.DMA,
    ],
    compiler_params=pltpu.CompilerParams(use_tc_tiling_on_sc=False),
)
def _deg_kernel(dst_hbm, ew_hbm, out_hbm, deg_sh, dstv, ewv, buf, sem):
    cid = lax.axis_index("c")
    sid = lax.axis_index("s")
    wid = sid * NC + cid

    def zb(i, _):
        buf[pl.ds(i * 16, 16)] = jnp.zeros((16,), jnp.float32)
        return 0
    lax.fori_loop(0, RPT // 16, zb, 0)
    pltpu.sync_copy(buf, deg_sh.at[pl.ds(sid * RPT, RPT)])
    plsc.subcore_barrier()

    pltpu.sync_copy(dst_hbm.at[pl.ds(wid * NCHUNK, NCHUNK)], dstv)
    pltpu.sync_copy(ew_hbm.at[pl.ds(wid * NCHUNK, NCHUNK)], ewv)

    def fire(i, _):
        pltpu.sync_copy(ewv.at[i], deg_sh.at[dstv.at[i]], add=True)
        return 0
    lax.fori_loop(0, NCHUNK, fire, 0)
    plsc.subcore_barrier()

    pltpu.sync_copy(deg_sh.at[pl.ds(sid * RPT, RPT)], buf)
    pltpu.sync_copy(buf, out_hbm.at[cid, pl.ds(sid * RPT, RPT)])


@functools.partial(
    pl.kernel,
    out_type=jax.ShapeDtypeStruct((NC, NP, H), jnp.float32),
    mesh=plsc.VectorSubcoreMesh(**_MESH),
    scratch_types=[
        pltpu.VMEM_SHARED((NP, H), jnp.float32),
        pltpu.VMEM((NCHUNK, CH), jnp.int32),
        pltpu.VMEM((NCHUNK, CH), jnp.int32),
        pltpu.VMEM((NCHUNK, CH), jnp.float32),
        pltpu.VMEM((CH, H), jnp.float32),
        pltpu.VMEM((CH, H), jnp.float32),
        pltpu.VMEM((CH, H), jnp.float32),
        pltpu.SemaphoreType.DMA,
        pltpu.SemaphoreType.DMA,
        pltpu.SemaphoreType.DMA,
        pltpu.SemaphoreType.DMA,
        pltpu.SemaphoreType.DMA,
        pltpu.SemaphoreType.DMA,
    ],
    compiler_params=pltpu.CompilerParams(use_tc_tiling_on_sc=False),
)
def _agg_kernel(tab_hbm, src_hbm, dst_hbm, ew_hbm, out_hbm,
                acc_sh, srcv, dstv, ewv, r0, r1, r2,
                g0, g1, g2, s0, s1, s2):
    cid = lax.axis_index("c")
    sid = lax.axis_index("s")
    wid = sid * NC + cid
    rows = (r0, r1, r2)
    gsem = (g0, g1, g2)
    ssem = (s0, s1, s2)

    def zb(i, _):
        r0[i // (H // 16), pl.ds((i % (H // 16)) * 16, 16)] = (
            jnp.zeros((16,), jnp.float32))
        return 0
    lax.fori_loop(0, CH * (H // 16), zb, 0)

    def zc(j, _):
        pltpu.sync_copy(r0, acc_sh.at[pl.ds(sid * RPT + j * CH, CH)])
        return 0
    lax.fori_loop(0, RPT // CH, zc, 0)
    plsc.subcore_barrier()

    pltpu.sync_copy(src_hbm.at[pl.ds(wid * NCHUNK, NCHUNK)], srcv)
    pltpu.sync_copy(dst_hbm.at[pl.ds(wid * NCHUNK, NCHUNK)], dstv)
    pltpu.sync_copy(ew_hbm.at[pl.ds(wid * NCHUNK, NCHUNK)], ewv)

    pltpu.async_copy(tab_hbm.at[srcv.at[0]], r0, g0)
    pltpu.async_copy(tab_hbm.at[srcv.at[1]], r1, g1)

    def outer(h, _):
        for b in range(3):
            i = h * 3 + b
            bp = (b + 2) % 3
            rb, gb, sb = rows[b], gsem[b], ssem[b]
            pltpu.make_async_copy(tab_hbm.at[srcv.at[i]], rb, gb).wait()

            @pl.when(i >= 1)
            def _():
                pltpu.make_async_copy(
                    rows[bp], acc_sh.at[dstv.at[i - 1]], ssem[bp]).wait()

            @pl.when(i + 2 < NCHUNK)
            def _():
                pltpu.async_copy(tab_hbm.at[srcv.at[i + 2]], rows[bp],
                                 gsem[bp])

            def scale(g, _):
                ew16 = ewv[i, pl.ds(g * 16, 16)]
                for j in range(16):
                    e = g * 16 + j
                    s = ew16[j]
                    for f in range(H // 16):
                        rb[e, pl.ds(f * 16, 16)] = (
                            rb[e, pl.ds(f * 16, 16)] * s)
                return 0
            lax.fori_loop(0, CH // 16, scale, 0)

            pltpu.async_copy(rb, acc_sh.at[dstv.at[i]], sb, add=True)
        return 0
    lax.fori_loop(0, NCHUNK // 3, outer, 0)

    pltpu.make_async_copy(
        rows[(NCHUNK - 1) % 3], acc_sh.at[dstv.at[NCHUNK - 1]],
        ssem[(NCHUNK - 1) % 3]).wait()
    plsc.subcore_barrier()

    def dump(j, _):
        pltpu.sync_copy(acc_sh.at[pl.ds(sid * RPT + j * CH, CH)], r0)
        pltpu.sync_copy(r0, out_hbm.at[cid, pl.ds(sid * RPT + j * CH, CH)])
        return 0
    lax.fori_loop(0, RPT // CH, dump, 0)


BR = 1024
GRID = NP // BR


def _dinv(dga, dgb):
    deg = dga + dgb
    return jnp.where(deg > 0, lax.rsqrt(deg), 0.0)


def _mm1_body(x_ref, w_ref, dga_ref, dgb_ref, o_ref):
    dinv = _dinv(dga_ref[...], dgb_ref[...])
    o_ref[...] = jnp.dot(x_ref[...], w_ref[...],
                         preferred_element_type=jnp.float32) * dinv


_mm1 = pl.pallas_call(
    _mm1_body,
    grid=(GRID,),
    in_specs=[
        pl.BlockSpec((BR, D), lambda i: (i, 0)),
        pl.BlockSpec((D, H), lambda i: (0, 0)),
        pl.BlockSpec((BR, 1), lambda i: (i, 0)),
        pl.BlockSpec((BR, 1), lambda i: (i, 0)),
    ],
    out_specs=pl.BlockSpec((BR, H), lambda i: (i, 0)),
    out_shape=jax.ShapeDtypeStruct((NP, H), jnp.float32),
)


def _mm2_body(a_ref, b_ref, dga_ref, dgb_ref, b1_ref, w_ref, o_ref):
    dinv = _dinv(dga_ref[...], dgb_ref[...])
    h = jnp.maximum((a_ref[...] + b_ref[...]) * dinv + b1_ref[...], 0.0)
    o_ref[...] = jnp.dot(h, w_ref[...],
                         preferred_element_type=jnp.float32) * dinv


_mm2 = pl.pallas_call(
    _mm2_body,
    grid=(GRID,),
    in_specs=[
        pl.BlockSpec((BR, H), lambda i: (i, 0)),
        pl.BlockSpec((BR, H), lambda i: (i, 0)),
        pl.BlockSpec((BR, 1), lambda i: (i, 0)),
        pl.BlockSpec((BR, 1), lambda i: (i, 0)),
        pl.BlockSpec((1, H), lambda i: (0, 0)),
        pl.BlockSpec((H, H), lambda i: (0, 0)),
    ],
    out_specs=pl.BlockSpec((BR, H), lambda i: (i, 0)),
    out_shape=jax.ShapeDtypeStruct((NP, H), jnp.float32),
)


def _pool_body(a_ref, b_ref, dga_ref, dgb_ref, b2_ref, bat_ref, wl_ref,
               bl_ref, out_ref, pooled_ref, sums, cnts):
    i = pl.program_id(0)

    @pl.when(i == 0)
    def _():
        sums[...] = jnp.zeros_like(sums)
        cnts[...] = jnp.zeros_like(cnts)

    dinv = _dinv(dga_ref[...], dgb_ref[...])
    h2 = jnp.maximum((a_ref[...] + b_ref[...]) * dinv + b2_ref[...], 0.0)
    sel = (bat_ref[...] == lax.broadcasted_iota(jnp.int32, (BR, G), 1)
           ).astype(jnp.float32)
    sums[...] += lax.dot_general(sel, h2, (((0,), (0,)), ((), ())),
                                 preferred_element_type=jnp.float32)
    cnts[...] += lax.dot_general(sel, jnp.ones((BR, 1), jnp.float32),
                                 (((0,), (0,)), ((), ())),
                                 preferred_element_type=jnp.float32)

    @pl.when(i == GRID - 1)
    def _():
        pooled = sums[...] / jnp.maximum(cnts[...], 1.0)
        pooled_ref[...] = pooled
        out_ref[...] = jnp.dot(pooled, wl_ref[...],
                               preferred_element_type=jnp.float32) + bl_ref[...]


_pool = pl.pallas_call(
    _pool_body,
    grid=(GRID,),
    in_specs=[
        pl.BlockSpec((BR, H), lambda i: (i, 0)),
        pl.BlockSpec((BR, H), lambda i: (i, 0)),
        pl.BlockSpec((BR, 1), lambda i: (i, 0)),
        pl.BlockSpec((BR, 1), lambda i: (i, 0)),
        pl.BlockSpec((1, H), lambda i: (0, 0)),
        pl.BlockSpec((BR, 1), lambda i: (i, 0)),
        pl.BlockSpec((H, C), lambda i: (0, 0)),
        pl.BlockSpec((1, C), lambda i: (0, 0)),
    ],
    out_specs=[
        pl.BlockSpec((G, C), lambda i: (0, 0)),
        pl.BlockSpec((G, H), lambda i: (0, 0)),
    ],
    out_shape=[
        jax.ShapeDtypeStruct((G, C), jnp.float32),
        jax.ShapeDtypeStruct((G, H), jnp.float32),
    ],
    scratch_shapes=[
        pltpu.VMEM((G, H), jnp.float32),
        pltpu.VMEM((G, 1), jnp.float32),
    ],
)


def kernel(x, edge_index, edge_weight, batch, W1, b1, W2, b2, Wl, bl):
    src, dst = edge_index[0], edge_index[1]
    loop_idx = jnp.arange(N, dtype=src.dtype)
    srcf = jnp.concatenate([src, loop_idx])
    dstf = jnp.concatenate([dst, loop_idx])
    ewf = jnp.concatenate([edge_weight, jnp.ones((N,), edge_weight.dtype)])

    pad = EP - EF
    srcf = jnp.pad(srcf, (0, pad)).reshape(EP // CH, CH)
    dstf = jnp.pad(dstf, (0, pad)).reshape(EP // CH, CH)
    ewf = jnp.pad(ewf, (0, pad)).reshape(EP // CH, CH)
    xp = jnp.pad(x, ((0, NP - N), (0, 0)))
    batp = jnp.pad(batch, (0, NP - N), constant_values=G).reshape(NP, 1)

    deg2 = _deg_kernel(dstf, ewf)
    dga = deg2[0].reshape(NP, 1)
    dgb = deg2[1].reshape(NP, 1)

    xs1 = _mm1(xp, W1, dga, dgb)
    acc1 = _agg_kernel(xs1, srcf, dstf, ewf)
    xs2 = _mm2(acc1[0], acc1[1], dga, dgb, b1.reshape(1, H), W2)
    acc2 = _agg_kernel(xs2, srcf, dstf, ewf)
    out, pooled = _pool(acc2[0], acc2[1], dga, dgb, b2.reshape(1, H),
                        batp, Wl, bl.reshape(1, C))
    return (out, pooled)


# speedup vs baseline: 26.3745x; 1.4643x over previous
import functools

import jax
import jax.numpy as jnp
from jax import lax
from jax.experimental import pallas as pl
from jax.experimental.pallas import tpu as pltpu
from jax.experimental.pallas import tpu_sc as plsc

N = 10000
D = 128
H = 64
C = 2
G = 64
E = 320000

NC = 2
NS = 16
NW = NC * NS

NP = 10240
RPT = NP // NS
CH = 128
EF = E + N
_NC0 = (EF + NW * CH - 1) // (NW * CH)
NCHUNK = ((_NC0 + 2) // 3) * 3
ET = NCHUNK * CH
EP = NW * ET

_MESH = dict(core_axis_name="c", subcore_axis_name="s",
             num_cores=NC, num_subcores=NS)


@functools.partial(
    pl.kernel,
    out_type=jax.ShapeDtypeStruct((NC, NP), jnp.float32),
    mesh=plsc.VectorSubcoreMesh(**_MESH),
    scratch_types=[
        pltpu.VMEM_SHARED((NP,), jnp.float32),
        pltpu.VMEM((NCHUNK, CH), jnp.int32),
        pltpu.VMEM((NCHUNK, CH), jnp.float32),
        pltpu.VMEM((RPT,), jnp.float32),
        pltpu.SemaphoreType.DMA,
    ],
    compiler_params=pltpu.CompilerParams(use_tc_tiling_on_sc=False),
)
def _deg_kernel(dst_hbm, ew_hbm, out_hbm, deg_sh, dstv, ewv, buf, sem):
    cid = lax.axis_index("c")
    sid = lax.axis_index("s")
    wid = sid * NC + cid

    def zb(i, _):
        buf[pl.ds(i * 16, 16)] = jnp.zeros((16,), jnp.float32)
        return 0
    lax.fori_loop(0, RPT // 16, zb, 0)
    pltpu.sync_copy(buf, deg_sh.at[pl.ds(sid * RPT, RPT)])
    plsc.subcore_barrier()

    pltpu.sync_copy(dst_hbm.at[pl.ds(wid * NCHUNK, NCHUNK)], dstv)
    pltpu.sync_copy(ew_hbm.at[pl.ds(wid * NCHUNK, NCHUNK)], ewv)

    def fire(i, _):
        pltpu.sync_copy(ewv.at[i], deg_sh.at[dstv.at[i]], add=True)
        return 0
    lax.fori_loop(0, NCHUNK, fire, 0)
    plsc.subcore_barrier()

    pltpu.sync_copy(deg_sh.at[pl.ds(sid * RPT, RPT)], buf)
    pltpu.sync_copy(buf, out_hbm.at[cid, pl.ds(sid * RPT, RPT)])


@functools.partial(
    pl.kernel,
    out_type=jax.ShapeDtypeStruct((NC, NP, H), jnp.float32),
    mesh=plsc.VectorSubcoreMesh(**_MESH),
    scratch_types=[
        pltpu.VMEM_SHARED((NP, H), jnp.float32),
        pltpu.VMEM((NCHUNK, CH), jnp.int32),
        pltpu.VMEM((NCHUNK, CH), jnp.int32),
        pltpu.VMEM((NCHUNK, CH), jnp.float32),
        pltpu.VMEM((CH, H), jnp.float32),
        pltpu.VMEM((CH, H), jnp.float32),
        pltpu.VMEM((CH, H), jnp.float32),
        pltpu.SemaphoreType.DMA,
        pltpu.SemaphoreType.DMA,
        pltpu.SemaphoreType.DMA,
        pltpu.SemaphoreType.DMA,
        pltpu.SemaphoreType.DMA,
        pltpu.SemaphoreType.DMA,
    ],
    compiler_params=pltpu.CompilerParams(use_tc_tiling_on_sc=False),
)
def _agg_kernel(tab_hbm, src_hbm, dst_hbm, ew_hbm, out_hbm,
                acc_sh, srcv, dstv, ewv, r0, r1, r2,
                g0, g1, g2, s0, s1, s2):
    cid = lax.axis_index("c")
    sid = lax.axis_index("s")
    wid = sid * NC + cid
    rows = (r0, r1, r2)
    gsem = (g0, g1, g2)
    ssem = (s0, s1, s2)

    def zb(i, _):
        r0[i // (H // 16), pl.ds((i % (H // 16)) * 16, 16)] = (
            jnp.zeros((16,), jnp.float32))
        return 0
    lax.fori_loop(0, CH * (H // 16), zb, 0)

    def zc(j, _):
        pltpu.sync_copy(r0, acc_sh.at[pl.ds(sid * RPT + j * CH, CH)])
        return 0
    lax.fori_loop(0, RPT // CH, zc, 0)
    plsc.subcore_barrier()

    pltpu.sync_copy(src_hbm.at[pl.ds(wid * NCHUNK, NCHUNK)], srcv)
    pltpu.sync_copy(dst_hbm.at[pl.ds(wid * NCHUNK, NCHUNK)], dstv)
    pltpu.sync_copy(ew_hbm.at[pl.ds(wid * NCHUNK, NCHUNK)], ewv)

    pltpu.async_copy(tab_hbm.at[srcv.at[0]], r0, g0)
    pltpu.async_copy(tab_hbm.at[srcv.at[1]], r1, g1)

    def outer(h, _):
        for b in range(3):
            i = h * 3 + b
            bp = (b + 2) % 3
            rb, gb, sb = rows[b], gsem[b], ssem[b]
            pltpu.make_async_copy(tab_hbm.at[srcv.at[i]], rb, gb).wait()

            @pl.when(i >= 1)
            def _():
                pltpu.make_async_copy(
                    rows[bp], acc_sh.at[dstv.at[i - 1]], ssem[bp]).wait()

            @pl.when(i + 2 < NCHUNK)
            def _():
                pltpu.async_copy(tab_hbm.at[srcv.at[i + 2]], rows[bp],
                                 gsem[bp])

            def scale(g, _):
                ew16 = ewv[i, pl.ds(g * 16, 16)]
                for j in range(16):
                    e = g * 16 + j
                    s = ew16[j]
                    vals = [rb[e, pl.ds(f * 16, 16)] * s
                            for f in range(H // 16)]
                    for f in range(H // 16):
                        rb[e, pl.ds(f * 16, 16)] = vals[f]
                return 0
            lax.fori_loop(0, CH // 16, scale, 0)

            pltpu.async_copy(rb, acc_sh.at[dstv.at[i]], sb, add=True)
        return 0
    lax.fori_loop(0, NCHUNK // 3, outer, 0)

    pltpu.make_async_copy(
        rows[(NCHUNK - 1) % 3], acc_sh.at[dstv.at[NCHUNK - 1]],
        ssem[(NCHUNK - 1) % 3]).wait()
    plsc.subcore_barrier()

    def dump(j, _):
        pltpu.sync_copy(acc_sh.at[pl.ds(sid * RPT + j * CH, CH)], r0)
        pltpu.sync_copy(r0, out_hbm.at[cid, pl.ds(sid * RPT + j * CH, CH)])
        return 0
    lax.fori_loop(0, RPT // CH, dump, 0)


BR = 1024
GRID = NP // BR


def _dinv(dga, dgb):
    deg = dga + dgb
    return jnp.where(deg > 0, lax.rsqrt(deg), 0.0)


def _mm1_body(x_ref, w_ref, dga_ref, dgb_ref, o_ref):
    dinv = _dinv(dga_ref[...], dgb_ref[...])
    o_ref[...] = jnp.dot(x_ref[...], w_ref[...],
                         preferred_element_type=jnp.float32) * dinv


_mm1 = pl.pallas_call(
    _mm1_body,
    grid=(GRID,),
    in_specs=[
        pl.BlockSpec((BR, D), lambda i: (i, 0)),
        pl.BlockSpec((D, H), lambda i: (0, 0)),
        pl.BlockSpec((BR, 1), lambda i: (i, 0)),
        pl.BlockSpec((BR, 1), lambda i: (i, 0)),
    ],
    out_specs=pl.BlockSpec((BR, H), lambda i: (i, 0)),
    out_shape=jax.ShapeDtypeStruct((NP, H), jnp.float32),
)


def _mm2_body(a_ref, b_ref, dga_ref, dgb_ref, b1_ref, w_ref, o_ref):
    dinv = _dinv(dga_ref[...], dgb_ref[...])
    h = jnp.maximum((a_ref[...] + b_ref[...]) * dinv + b1_ref[...], 0.0)
    o_ref[...] = jnp.dot(h, w_ref[...],
                         preferred_element_type=jnp.float32) * dinv


_mm2 = pl.pallas_call(
    _mm2_body,
    grid=(GRID,),
    in_specs=[
        pl.BlockSpec((BR, H), lambda i: (i, 0)),
        pl.BlockSpec((BR, H), lambda i: (i, 0)),
        pl.BlockSpec((BR, 1), lambda i: (i, 0)),
        pl.BlockSpec((BR, 1), lambda i: (i, 0)),
        pl.BlockSpec((1, H), lambda i: (0, 0)),
        pl.BlockSpec((H, H), lambda i: (0, 0)),
    ],
    out_specs=pl.BlockSpec((BR, H), lambda i: (i, 0)),
    out_shape=jax.ShapeDtypeStruct((NP, H), jnp.float32),
)


def _pool_body(a_ref, b_ref, dga_ref, dgb_ref, b2_ref, bat_ref, wl_ref,
               bl_ref, out_ref, pooled_ref, sums, cnts):
    i = pl.program_id(0)

    @pl.when(i == 0)
    def _():
        sums[...] = jnp.zeros_like(sums)
        cnts[...] = jnp.zeros_like(cnts)

    dinv = _dinv(dga_ref[...], dgb_ref[...])
    h2 = jnp.maximum((a_ref[...] + b_ref[...]) * dinv + b2_ref[...], 0.0)
    sel = (bat_ref[...] == lax.broadcasted_iota(jnp.int32, (BR, G), 1)
           ).astype(jnp.float32)
    sums[...] += lax.dot_general(sel, h2, (((0,), (0,)), ((), ())),
                                 preferred_element_type=jnp.float32)
    cnts[...] += lax.dot_general(sel, jnp.ones((BR, 1), jnp.float32),
                                 (((0,), (0,)), ((), ())),
                                 preferred_element_type=jnp.float32)

    @pl.when(i == GRID - 1)
    def _():
        pooled = sums[...] / jnp.maximum(cnts[...], 1.0)
        pooled_ref[...] = pooled
        out_ref[...] = jnp.dot(pooled, wl_ref[...],
                               preferred_element_type=jnp.float32) + bl_ref[...]


_pool = pl.pallas_call(
    _pool_body,
    grid=(GRID,),
    in_specs=[
        pl.BlockSpec((BR, H), lambda i: (i, 0)),
        pl.BlockSpec((BR, H), lambda i: (i, 0)),
        pl.BlockSpec((BR, 1), lambda i: (i, 0)),
        pl.BlockSpec((BR, 1), lambda i: (i, 0)),
        pl.BlockSpec((1, H), lambda i: (0, 0)),
        pl.BlockSpec((BR, 1), lambda i: (i, 0)),
        pl.BlockSpec((H, C), lambda i: (0, 0)),
        pl.BlockSpec((1, C), lambda i: (0, 0)),
    ],
    out_specs=[
        pl.BlockSpec((G, C), lambda i: (0, 0)),
        pl.BlockSpec((G, H), lambda i: (0, 0)),
    ],
    out_shape=[
        jax.ShapeDtypeStruct((G, C), jnp.float32),
        jax.ShapeDtypeStruct((G, H), jnp.float32),
    ],
    scratch_shapes=[
        pltpu.VMEM((G, H), jnp.float32),
        pltpu.VMEM((G, 1), jnp.float32),
    ],
)


def kernel(x, edge_index, edge_weight, batch, W1, b1, W2, b2, Wl, bl):
    src, dst = edge_index[0], edge_index[1]
    loop_idx = jnp.arange(N, dtype=src.dtype)
    srcf = jnp.concatenate([src, loop_idx])
    dstf = jnp.concatenate([dst, loop_idx])
    ewf = jnp.concatenate([edge_weight, jnp.ones((N,), edge_weight.dtype)])

    pad = EP - EF
    srcf = jnp.pad(srcf, (0, pad)).reshape(EP // CH, CH)
    dstf = jnp.pad(dstf, (0, pad)).reshape(EP // CH, CH)
    ewf = jnp.pad(ewf, (0, pad)).reshape(EP // CH, CH)
    xp = jnp.pad(x, ((0, NP - N), (0, 0)))
    batp = jnp.pad(batch, (0, NP - N), constant_values=G).reshape(NP, 1)

    deg2 = _deg_kernel(dstf, ewf)
    dga = deg2[0].reshape(NP, 1)
    dgb = deg2[1].reshape(NP, 1)

    xs1 = _mm1(xp, W1, dga, dgb)
    acc1 = _agg_kernel(xs1, srcf, dstf, ewf)
    xs2 = _mm2(acc1[0], acc1[1], dga, dgb, b1.reshape(1, H), W2)
    acc2 = _agg_kernel(xs2, srcf, dstf, ewf)
    out, pooled = _pool(acc2[0], acc2[1], dga, dgb, b2.reshape(1, H),
                        batp, Wl, bl.reshape(1, C))
    return (out, pooled)


# speedup vs baseline: 29.7349x; 1.1274x over previous
import functools

import jax
import jax.numpy as jnp
from jax import lax
from jax.experimental import pallas as pl
from jax.experimental.pallas import tpu as pltpu
from jax.experimental.pallas import tpu_sc as plsc

N = 10000
D = 128
H = 64
C = 2
G = 64
E = 320000

NC = 2
NS = 16
NW = NC * NS

NP = 10240
RPT = NP // NS
CH = 128
EF = E + N
_NC0 = (EF + NW * CH - 1) // (NW * CH)
NCHUNK = ((_NC0 + 2) // 3) * 3
ET = NCHUNK * CH
EP = NW * ET

_MESH = dict(core_axis_name="c", subcore_axis_name="s",
             num_cores=NC, num_subcores=NS)


@functools.partial(
    pl.kernel,
    out_type=jax.ShapeDtypeStruct((NC, NP), jnp.float32),
    mesh=plsc.VectorSubcoreMesh(**_MESH),
    scratch_types=[
        pltpu.VMEM_SHARED((NP,), jnp.float32),
        pltpu.VMEM((NCHUNK, CH), jnp.int32),
        pltpu.VMEM((NCHUNK, CH), jnp.float32),
        pltpu.VMEM((RPT,), jnp.float32),
        pltpu.SemaphoreType.DMA,
    ],
    compiler_params=pltpu.CompilerParams(use_tc_tiling_on_sc=False),
)
def _deg_kernel(dst_hbm, ew_hbm, out_hbm, deg_sh, dstv, ewv, buf, sem):
    cid = lax.axis_index("c")
    sid = lax.axis_index("s")
    wid = sid * NC + cid

    def zb(i, _):
        buf[pl.ds(i * 16, 16)] = jnp.zeros((16,), jnp.float32)
        return 0
    lax.fori_loop(0, RPT // 16, zb, 0)
    pltpu.sync_copy(buf, deg_sh.at[pl.ds(sid * RPT, RPT)])
    plsc.subcore_barrier()

    pltpu.sync_copy(dst_hbm.at[pl.ds(wid * NCHUNK, NCHUNK)], dstv)
    pltpu.sync_copy(ew_hbm.at[pl.ds(wid * NCHUNK, NCHUNK)], ewv)

    def fire(i, _):
        pltpu.sync_copy(ewv.at[i], deg_sh.at[dstv.at[i]], add=True)
        return 0
    lax.fori_loop(0, NCHUNK, fire, 0)
    plsc.subcore_barrier()

    pltpu.sync_copy(deg_sh.at[pl.ds(sid * RPT, RPT)], buf)
    pltpu.sync_copy(buf, out_hbm.at[cid, pl.ds(sid * RPT, RPT)])


@functools.partial(
    pl.kernel,
    out_type=jax.ShapeDtypeStruct((NC, NP, H), jnp.float32),
    mesh=plsc.VectorSubcoreMesh(**_MESH),
    scratch_types=[
        pltpu.VMEM_SHARED((NP, H), jnp.float32),
        pltpu.VMEM((NCHUNK, CH), jnp.int32),
        pltpu.VMEM((NCHUNK, CH), jnp.int32),
        pltpu.VMEM((NCHUNK, CH), jnp.float32),
        pltpu.VMEM((CH, H), jnp.float32),
        pltpu.VMEM((CH, H), jnp.float32),
        pltpu.VMEM((CH, H), jnp.float32),
        pltpu.SemaphoreType.DMA,
        pltpu.SemaphoreType.DMA,
        pltpu.SemaphoreType.DMA,
        pltpu.SemaphoreType.DMA,
        pltpu.SemaphoreType.DMA,
        pltpu.SemaphoreType.DMA,
    ],
    compiler_params=pltpu.CompilerParams(use_tc_tiling_on_sc=False),
)
def _agg_kernel(tab_hbm, src_hbm, dst_hbm, ew_hbm, out_hbm,
                acc_sh, srcv, dstv, ewv, r0, r1, r2,
                g0, g1, g2, s0, s1, s2):
    cid = lax.axis_index("c")
    sid = lax.axis_index("s")
    wid = sid * NC + cid
    rows = (r0, r1, r2)
    gsem = (g0, g1, g2)
    ssem = (s0, s1, s2)

    def zb(i, _):
        r0[i // (H // 16), pl.ds((i % (H // 16)) * 16, 16)] = (
            jnp.zeros((16,), jnp.float32))
        return 0
    lax.fori_loop(0, CH * (H // 16), zb, 0)

    def zc(j, _):
        pltpu.sync_copy(r0, acc_sh.at[pl.ds(sid * RPT + j * CH, CH)])
        return 0
    lax.fori_loop(0, RPT // CH, zc, 0)
    plsc.subcore_barrier()

    pltpu.sync_copy(src_hbm.at[pl.ds(wid * NCHUNK, NCHUNK)], srcv)
    pltpu.sync_copy(dst_hbm.at[pl.ds(wid * NCHUNK, NCHUNK)], dstv)
    pltpu.sync_copy(ew_hbm.at[pl.ds(wid * NCHUNK, NCHUNK)], ewv)

    pltpu.async_copy(tab_hbm.at[srcv.at[0]], r0, g0)
    pltpu.async_copy(tab_hbm.at[srcv.at[1]], r1, g1)

    def outer(h, _):
        for b in range(3):
            i = h * 3 + b
            bp = (b + 2) % 3
            rb, gb, sb = rows[b], gsem[b], ssem[b]
            pltpu.make_async_copy(tab_hbm.at[srcv.at[i]], rb, gb).wait()

            @pl.when(i >= 1)
            def _():
                pltpu.make_async_copy(
                    rows[bp], acc_sh.at[dstv.at[i - 1]], ssem[bp]).wait()

            @pl.when(i + 2 < NCHUNK)
            def _():
                pltpu.async_copy(tab_hbm.at[srcv.at[i + 2]], rows[bp],
                                 gsem[bp])

            def scale(g, _):
                ew16 = ewv[i, pl.ds(g * 16, 16)]
                for j in range(0, 16, 4):
                    e = g * 16 + j
                    vals = []
                    for k in range(4):
                        s = ew16[j + k]
                        vals.append([rb[e + k, pl.ds(f * 16, 16)] * s
                                     for f in range(H // 16)])
                    for k in range(4):
                        for f in range(H // 16):
                            rb[e + k, pl.ds(f * 16, 16)] = vals[k][f]
                return 0
            lax.fori_loop(0, CH // 16, scale, 0)

            pltpu.async_copy(rb, acc_sh.at[dstv.at[i]], sb, add=True)
        return 0
    lax.fori_loop(0, NCHUNK // 3, outer, 0)

    pltpu.make_async_copy(
        rows[(NCHUNK - 1) % 3], acc_sh.at[dstv.at[NCHUNK - 1]],
        ssem[(NCHUNK - 1) % 3]).wait()
    plsc.subcore_barrier()

    def dump(j, _):
        pltpu.sync_copy(acc_sh.at[pl.ds(sid * RPT + j * CH, CH)], r0)
        pltpu.sync_copy(r0, out_hbm.at[cid, pl.ds(sid * RPT + j * CH, CH)])
        return 0
    lax.fori_loop(0, RPT // CH, dump, 0)


BR = 1024
GRID = NP // BR


def _dinv(dga, dgb):
    deg = dga + dgb
    return jnp.where(deg > 0, lax.rsqrt(deg), 0.0)


def _mm1_body(x_ref, w_ref, dga_ref, dgb_ref, o_ref):
    dinv = _dinv(dga_ref[...], dgb_ref[...])
    o_ref[...] = jnp.dot(x_ref[...], w_ref[...],
                         preferred_element_type=jnp.float32) * dinv


_mm1 = pl.pallas_call(
    _mm1_body,
    grid=(GRID,),
    in_specs=[
        pl.BlockSpec((BR, D), lambda i: (i, 0)),
        pl.BlockSpec((D, H), lambda i: (0, 0)),
        pl.BlockSpec((BR, 1), lambda i: (i, 0)),
        pl.BlockSpec((BR, 1), lambda i: (i, 0)),
    ],
    out_specs=pl.BlockSpec((BR, H), lambda i: (i, 0)),
    out_shape=jax.ShapeDtypeStruct((NP, H), jnp.float32),
)


def _mm2_body(a_ref, b_ref, dga_ref, dgb_ref, b1_ref, w_ref, o_ref):
    dinv = _dinv(dga_ref[...], dgb_ref[...])
    h = jnp.maximum((a_ref[...] + b_ref[...]) * dinv + b1_ref[...], 0.0)
    o_ref[...] = jnp.dot(h, w_ref[...],
                         preferred_element_type=jnp.float32) * dinv


_mm2 = pl.pallas_call(
    _mm2_body,
    grid=(GRID,),
    in_specs=[
        pl.BlockSpec((BR, H), lambda i: (i, 0)),
        pl.BlockSpec((BR, H), lambda i: (i, 0)),
        pl.BlockSpec((BR, 1), lambda i: (i, 0)),
        pl.BlockSpec((BR, 1), lambda i: (i, 0)),
        pl.BlockSpec((1, H), lambda i: (0, 0)),
        pl.BlockSpec((H, H), lambda i: (0, 0)),
    ],
    out_specs=pl.BlockSpec((BR, H), lambda i: (i, 0)),
    out_shape=jax.ShapeDtypeStruct((NP, H), jnp.float32),
)


def _pool_body(a_ref, b_ref, dga_ref, dgb_ref, b2_ref, bat_ref, wl_ref,
               bl_ref, out_ref, pooled_ref, sums, cnts):
    i = pl.program_id(0)

    @pl.when(i == 0)
    def _():
        sums[...] = jnp.zeros_like(sums)
        cnts[...] = jnp.zeros_like(cnts)

    dinv = _dinv(dga_ref[...], dgb_ref[...])
    h2 = jnp.maximum((a_ref[...] + b_ref[...]) * dinv + b2_ref[...], 0.0)
    sel = (bat_ref[...] == lax.broadcasted_iota(jnp.int32, (BR, G), 1)
           ).astype(jnp.float32)
    sums[...] += lax.dot_general(sel, h2, (((0,), (0,)), ((), ())),
                                 preferred_element_type=jnp.float32)
    cnts[...] += lax.dot_general(sel, jnp.ones((BR, 1), jnp.float32),
                                 (((0,), (0,)), ((), ())),
                                 preferred_element_type=jnp.float32)

    @pl.when(i == GRID - 1)
    def _():
        pooled = sums[...] / jnp.maximum(cnts[...], 1.0)
        pooled_ref[...] = pooled
        out_ref[...] = jnp.dot(pooled, wl_ref[...],
                               preferred_element_type=jnp.float32) + bl_ref[...]


_pool = pl.pallas_call(
    _pool_body,
    grid=(GRID,),
    in_specs=[
        pl.BlockSpec((BR, H), lambda i: (i, 0)),
        pl.BlockSpec((BR, H), lambda i: (i, 0)),
        pl.BlockSpec((BR, 1), lambda i: (i, 0)),
        pl.BlockSpec((BR, 1), lambda i: (i, 0)),
        pl.BlockSpec((1, H), lambda i: (0, 0)),
        pl.BlockSpec((BR, 1), lambda i: (i, 0)),
        pl.BlockSpec((H, C), lambda i: (0, 0)),
        pl.BlockSpec((1, C), lambda i: (0, 0)),
    ],
    out_specs=[
        pl.BlockSpec((G, C), lambda i: (0, 0)),
        pl.BlockSpec((G, H), lambda i: (0, 0)),
    ],
    out_shape=[
        jax.ShapeDtypeStruct((G, C), jnp.float32),
        jax.ShapeDtypeStruct((G, H), jnp.float32),
    ],
    scratch_shapes=[
        pltpu.VMEM((G, H), jnp.float32),
        pltpu.VMEM((G, 1), jnp.float32),
    ],
)


def kernel(x, edge_index, edge_weight, batch, W1, b1, W2, b2, Wl, bl):
    src, dst = edge_index[0], edge_index[1]
    loop_idx = jnp.arange(N, dtype=src.dtype)
    srcf = jnp.concatenate([src, loop_idx])
    dstf = jnp.concatenate([dst, loop_idx])
    ewf = jnp.concatenate([edge_weight, jnp.ones((N,), edge_weight.dtype)])

    pad = EP - EF
    srcf = jnp.pad(srcf, (0, pad)).reshape(EP // CH, CH)
    dstf = jnp.pad(dstf, (0, pad)).reshape(EP // CH, CH)
    ewf = jnp.pad(ewf, (0, pad)).reshape(EP // CH, CH)
    xp = jnp.pad(x, ((0, NP - N), (0, 0)))
    batp = jnp.pad(batch, (0, NP - N), constant_values=G).reshape(NP, 1)

    deg2 = _deg_kernel(dstf, ewf)
    dga = deg2[0].reshape(NP, 1)
    dgb = deg2[1].reshape(NP, 1)

    xs1 = _mm1(xp, W1, dga, dgb)
    acc1 = _agg_kernel(xs1, srcf, dstf, ewf)
    xs2 = _mm2(acc1[0], acc1[1], dga, dgb, b1.reshape(1, H), W2)
    acc2 = _agg_kernel(xs2, srcf, dstf, ewf)
    out, pooled = _pool(acc2[0], acc2[1], dga, dgb, b2.reshape(1, H),
                        batp, Wl, bl.reshape(1, C))
    return (out, pooled)


# speedup vs baseline: 30.1036x; 1.0124x over previous
import functools

import jax
import jax.numpy as jnp
from jax import lax
from jax.experimental import pallas as pl
from jax.experimental.pallas import tpu as pltpu
from jax.experimental.pallas import tpu_sc as plsc

N = 10000
D = 128
H = 64
C = 2
G = 64
E = 320000

NC = 2
NS = 16
NW = NC * NS

NP = 10240
RPT = NP // NS
CH = 128
EF = E + N
_NC0 = (EF + NW * CH - 1) // (NW * CH)
NCHUNK = ((_NC0 + 2) // 3) * 3
ET = NCHUNK * CH
EP = NW * ET

_MESH = dict(core_axis_name="c", subcore_axis_name="s",
             num_cores=NC, num_subcores=NS)


@functools.partial(
    pl.kernel,
    out_type=jax.ShapeDtypeStruct((NC, NP), jnp.float32),
    mesh=plsc.VectorSubcoreMesh(**_MESH),
    scratch_types=[
        pltpu.VMEM_SHARED((NP,), jnp.float32),
        pltpu.VMEM((NCHUNK, CH), jnp.int32),
        pltpu.VMEM((NCHUNK, CH), jnp.float32),
        pltpu.VMEM((RPT,), jnp.float32),
        pltpu.SemaphoreType.DMA,
    ],
    compiler_params=pltpu.CompilerParams(use_tc_tiling_on_sc=False),
)
def _deg_kernel(dst_hbm, ew_hbm, out_hbm, deg_sh, dstv, ewv, buf, sem):
    cid = lax.axis_index("c")
    sid = lax.axis_index("s")
    wid = sid * NC + cid

    def zb(i, _):
        buf[pl.ds(i * 16, 16)] = jnp.zeros((16,), jnp.float32)
        return 0
    lax.fori_loop(0, RPT // 16, zb, 0)
    pltpu.sync_copy(buf, deg_sh.at[pl.ds(sid * RPT, RPT)])
    plsc.subcore_barrier()

    pltpu.sync_copy(dst_hbm.at[pl.ds(wid * NCHUNK, NCHUNK)], dstv)
    pltpu.sync_copy(ew_hbm.at[pl.ds(wid * NCHUNK, NCHUNK)], ewv)

    def fire(i, _):
        pltpu.sync_copy(ewv.at[i], deg_sh.at[dstv.at[i]], add=True)
        return 0
    lax.fori_loop(0, NCHUNK, fire, 0)
    plsc.subcore_barrier()

    pltpu.sync_copy(deg_sh.at[pl.ds(sid * RPT, RPT)], buf)
    pltpu.sync_copy(buf, out_hbm.at[cid, pl.ds(sid * RPT, RPT)])


@functools.partial(
    pl.kernel,
    out_type=jax.ShapeDtypeStruct((NC, NP, H), jnp.float32),
    mesh=plsc.VectorSubcoreMesh(**_MESH),
    scratch_types=[
        pltpu.VMEM_SHARED((NP, H), jnp.float32),
        pltpu.VMEM((NCHUNK, CH), jnp.int32),
        pltpu.VMEM((NCHUNK, CH), jnp.int32),
        pltpu.VMEM((NCHUNK, CH), jnp.float32),
        pltpu.VMEM((CH, H), jnp.float32),
        pltpu.VMEM((CH, H), jnp.float32),
        pltpu.VMEM((CH, H), jnp.float32),
        pltpu.SemaphoreType.DMA,
        pltpu.SemaphoreType.DMA,
        pltpu.SemaphoreType.DMA,
        pltpu.SemaphoreType.DMA,
        pltpu.SemaphoreType.DMA,
        pltpu.SemaphoreType.DMA,
    ],
    compiler_params=pltpu.CompilerParams(use_tc_tiling_on_sc=False),
)
def _agg_kernel(tab_hbm, src_hbm, dst_hbm, ew_hbm, out_hbm,
                acc_sh, srcv, dstv, ewv, r0, r1, r2,
                g0, g1, g2, s0, s1, s2):
    cid = lax.axis_index("c")
    sid = lax.axis_index("s")
    wid = sid * NC + cid
    rows = (r0, r1, r2)
    gsem = (g0, g1, g2)
    ssem = (s0, s1, s2)

    def zb(i, _):
        r0[i // (H // 16), pl.ds((i % (H // 16)) * 16, 16)] = (
            jnp.zeros((16,), jnp.float32))
        return 0
    lax.fori_loop(0, CH * (H // 16), zb, 0)

    def zc(j, _):
        pltpu.sync_copy(r0, acc_sh.at[pl.ds(sid * RPT + j * CH, CH)])
        return 0
    lax.fori_loop(0, RPT // CH, zc, 0)
    plsc.subcore_barrier()

    pltpu.sync_copy(src_hbm.at[pl.ds(wid * NCHUNK, NCHUNK)], srcv)
    pltpu.sync_copy(dst_hbm.at[pl.ds(wid * NCHUNK, NCHUNK)], dstv)
    pltpu.sync_copy(ew_hbm.at[pl.ds(wid * NCHUNK, NCHUNK)], ewv)

    pltpu.async_copy(tab_hbm.at[srcv.at[0]], r0, g0)
    pltpu.async_copy(tab_hbm.at[srcv.at[1]], r1, g1)

    def outer(h, _):
        for b in range(3):
            i = h * 3 + b
            bp = (b + 2) % 3
            rb, gb, sb = rows[b], gsem[b], ssem[b]
            pltpu.make_async_copy(tab_hbm.at[srcv.at[i]], rb, gb).wait()

            @pl.when(i >= 1)
            def _():
                pltpu.make_async_copy(
                    rows[bp], acc_sh.at[dstv.at[i - 1]], ssem[bp]).wait()

            @pl.when(i + 2 < NCHUNK)
            def _():
                pltpu.async_copy(tab_hbm.at[srcv.at[i + 2]], rows[bp],
                                 gsem[bp])

            def scale(g, _):
                ew16 = ewv[i, pl.ds(g * 16, 16)]
                for j in range(0, 16, 4):
                    e = g * 16 + j
                    vals = []
                    for k in range(4):
                        s = ew16[j + k]
                        vals.append([rb[e + k, pl.ds(f * 16, 16)] * s
                                     for f in range(H // 16)])
                    for k in range(4):
                        for f in range(H // 16):
                            rb[e + k, pl.ds(f * 16, 16)] = vals[k][f]
                return 0
            lax.fori_loop(0, CH // 16, scale, 0)

            pltpu.async_copy(rb, acc_sh.at[dstv.at[i]], sb, add=True)
        return 0
    lax.fori_loop(0, NCHUNK // 3, outer, 0)

    pltpu.make_async_copy(
        rows[(NCHUNK - 1) % 3], acc_sh.at[dstv.at[NCHUNK - 1]],
        ssem[(NCHUNK - 1) % 3]).wait()
    plsc.subcore_barrier()

    def dump(j, _):
        pltpu.sync_copy(acc_sh.at[pl.ds(sid * RPT + j * CH, CH)], r0)
        pltpu.sync_copy(r0, out_hbm.at[cid, pl.ds(sid * RPT + j * CH, CH)])
        return 0
    lax.fori_loop(0, RPT // CH, dump, 0)


BR = 1024
GRID = NP // BR


def _dinv(dga, dgb):
    deg = dga + dgb
    return jnp.where(deg > 0, lax.rsqrt(deg), 0.0)


def _dinv2(dg_ref, dg2_ref):
    deg = dg_ref[0] + dg2_ref[0]
    return jnp.where(deg > 0, lax.rsqrt(deg), 0.0)


def _mm1_body(x_ref, w_ref, dg_ref, dg2_ref, o_ref):
    dinv = _dinv2(dg_ref, dg2_ref)
    o_ref[...] = jnp.dot(x_ref[...], w_ref[...],
                         preferred_element_type=jnp.float32) * dinv


_mm1 = pl.pallas_call(
    _mm1_body,
    grid=(GRID,),
    in_specs=[
        pl.BlockSpec((BR, D), lambda i: (i, 0)),
        pl.BlockSpec((D, H), lambda i: (0, 0)),
        pl.BlockSpec((1, BR, 1), lambda i: (0, i, 0)),
        pl.BlockSpec((1, BR, 1), lambda i: (1, i, 0)),
    ],
    out_specs=pl.BlockSpec((BR, H), lambda i: (i, 0)),
    out_shape=jax.ShapeDtypeStruct((NP, H), jnp.float32),
)


def _mm2_body(a_ref, b_ref, dg_ref, dg2_ref, b1_ref, w_ref, o_ref):
    dinv = _dinv2(dg_ref, dg2_ref)
    h = jnp.maximum((a_ref[0] + b_ref[0]) * dinv + b1_ref[...], 0.0)
    o_ref[...] = jnp.dot(h, w_ref[...],
                         preferred_element_type=jnp.float32) * dinv


_mm2 = pl.pallas_call(
    _mm2_body,
    grid=(GRID,),
    in_specs=[
        pl.BlockSpec((1, BR, H), lambda i: (0, i, 0)),
        pl.BlockSpec((1, BR, H), lambda i: (1, i, 0)),
        pl.BlockSpec((1, BR, 1), lambda i: (0, i, 0)),
        pl.BlockSpec((1, BR, 1), lambda i: (1, i, 0)),
        pl.BlockSpec((1, H), lambda i: (0, 0)),
        pl.BlockSpec((H, H), lambda i: (0, 0)),
    ],
    out_specs=pl.BlockSpec((BR, H), lambda i: (i, 0)),
    out_shape=jax.ShapeDtypeStruct((NP, H), jnp.float32),
)


def _pool_body(a_ref, b_ref, dg_ref, dg2_ref, b2_ref, bat_ref, wl_ref,
               bl_ref, out_ref, pooled_ref, sums, cnts):
    i = pl.program_id(0)

    @pl.when(i == 0)
    def _():
        sums[...] = jnp.zeros_like(sums)
        cnts[...] = jnp.zeros_like(cnts)

    dinv = _dinv2(dg_ref, dg2_ref)
    h2 = jnp.maximum((a_ref[0] + b_ref[0]) * dinv + b2_ref[...], 0.0)
    sel = (bat_ref[...] == lax.broadcasted_iota(jnp.int32, (BR, G), 1)
           ).astype(jnp.float32)
    sums[...] += lax.dot_general(sel, h2, (((0,), (0,)), ((), ())),
                                 preferred_element_type=jnp.float32)
    cnts[...] += lax.dot_general(sel, jnp.ones((BR, 1), jnp.float32),
                                 (((0,), (0,)), ((), ())),
                                 preferred_element_type=jnp.float32)

    @pl.when(i == GRID - 1)
    def _():
        pooled = sums[...] / jnp.maximum(cnts[...], 1.0)
        pooled_ref[...] = pooled
        out_ref[...] = jnp.dot(pooled, wl_ref[...],
                               preferred_element_type=jnp.float32) + bl_ref[...]


_pool = pl.pallas_call(
    _pool_body,
    grid=(GRID,),
    in_specs=[
        pl.BlockSpec((1, BR, H), lambda i: (0, i, 0)),
        pl.BlockSpec((1, BR, H), lambda i: (1, i, 0)),
        pl.BlockSpec((1, BR, 1), lambda i: (0, i, 0)),
        pl.BlockSpec((1, BR, 1), lambda i: (1, i, 0)),
        pl.BlockSpec((1, H), lambda i: (0, 0)),
        pl.BlockSpec((BR, 1), lambda i: (i, 0)),
        pl.BlockSpec((H, C), lambda i: (0, 0)),
        pl.BlockSpec((1, C), lambda i: (0, 0)),
    ],
    out_specs=[
        pl.BlockSpec((G, C), lambda i: (0, 0)),
        pl.BlockSpec((G, H), lambda i: (0, 0)),
    ],
    out_shape=[
        jax.ShapeDtypeStruct((G, C), jnp.float32),
        jax.ShapeDtypeStruct((G, H), jnp.float32),
    ],
    scratch_shapes=[
        pltpu.VMEM((G, H), jnp.float32),
        pltpu.VMEM((G, 1), jnp.float32),
    ],
)


def kernel(x, edge_index, edge_weight, batch, W1, b1, W2, b2, Wl, bl):
    src, dst = edge_index[0], edge_index[1]
    loop_idx = jnp.arange(N, dtype=src.dtype)
    srcf = jnp.concatenate([src, loop_idx])
    dstf = jnp.concatenate([dst, loop_idx])
    ewf = jnp.concatenate([edge_weight, jnp.ones((N,), edge_weight.dtype)])

    pad = EP - EF
    srcf = jnp.pad(srcf, (0, pad)).reshape(EP // CH, CH)
    dstf = jnp.pad(dstf, (0, pad)).reshape(EP // CH, CH)
    ewf = jnp.pad(ewf, (0, pad)).reshape(EP // CH, CH)
    xp = jnp.pad(x, ((0, NP - N), (0, 0)))
    batp = jnp.pad(batch, (0, NP - N), constant_values=G).reshape(NP, 1)

    deg3 = _deg_kernel(dstf, ewf).reshape(NC, NP, 1)

    xs1 = _mm1(xp, W1, deg3, deg3)
    acc1 = _agg_kernel(xs1, srcf, dstf, ewf)
    xs2 = _mm2(acc1, acc1, deg3, deg3, b1.reshape(1, H), W2)
    acc2 = _agg_kernel(xs2, srcf, dstf, ewf)
    out, pooled = _pool(acc2, acc2, deg3, deg3, b2.reshape(1, H),
                        batp, Wl, bl.reshape(1, C))
    return (out, pooled)


# speedup vs baseline: 30.6836x; 1.0193x over previous
import functools

import jax
import jax.numpy as jnp
from jax import lax
from jax.experimental import pallas as pl
from jax.experimental.pallas import tpu as pltpu
from jax.experimental.pallas import tpu_sc as plsc

N = 10000
D = 128
H = 64
C = 2
G = 64
E = 320000

NC = 2
NS = 16
NW = NC * NS

NP = 10240
RPT = NP // NS
CH = 128
EF = E + N
_NC0 = (EF + NW * CH - 1) // (NW * CH)
NCHUNK = ((_NC0 + 2) // 3) * 3
ET = NCHUNK * CH
EP = NW * ET

_MESH = dict(core_axis_name="c", subcore_axis_name="s",
             num_cores=NC, num_subcores=NS)


@functools.partial(
    pl.kernel,
    out_type=jax.ShapeDtypeStruct((NC, NP), jnp.float32),
    mesh=plsc.VectorSubcoreMesh(**_MESH),
    scratch_types=[
        pltpu.VMEM_SHARED((NP,), jnp.float32),
        pltpu.VMEM((NCHUNK, CH), jnp.int32),
        pltpu.VMEM((NCHUNK, CH), jnp.float32),
        pltpu.VMEM((RPT,), jnp.float32),
        pltpu.SemaphoreType.DMA,
    ],
    compiler_params=pltpu.CompilerParams(use_tc_tiling_on_sc=False),
)
def _deg_kernel(dst_hbm, ew_hbm, out_hbm, deg_sh, dstv, ewv, buf, sem):
    cid = lax.axis_index("c")
    sid = lax.axis_index("s")
    wid = sid * NC + cid

    def zb(i, _):
        buf[pl.ds(i * 16, 16)] = jnp.zeros((16,), jnp.float32)
        return 0
    lax.fori_loop(0, RPT // 16, zb, 0)
    pltpu.sync_copy(buf, deg_sh.at[pl.ds(sid * RPT, RPT)])
    plsc.subcore_barrier()

    pltpu.sync_copy(dst_hbm.at[pl.ds(wid * NCHUNK, NCHUNK)], dstv)
    pltpu.sync_copy(ew_hbm.at[pl.ds(wid * NCHUNK, NCHUNK)], ewv)

    def fire(i, _):
        pltpu.sync_copy(ewv.at[i], deg_sh.at[dstv.at[i]], add=True)
        return 0
    lax.fori_loop(0, NCHUNK, fire, 0)
    plsc.subcore_barrier()

    pltpu.sync_copy(deg_sh.at[pl.ds(sid * RPT, RPT)], buf)
    pltpu.sync_copy(buf, out_hbm.at[cid, pl.ds(sid * RPT, RPT)])


@functools.partial(
    pl.kernel,
    out_type=jax.ShapeDtypeStruct((NC, NP, H), jnp.float32),
    mesh=plsc.VectorSubcoreMesh(**_MESH),
    scratch_types=[
        pltpu.VMEM_SHARED((NP, H), jnp.float32),
        pltpu.VMEM((NCHUNK, CH), jnp.int32),
        pltpu.VMEM((NCHUNK, CH), jnp.int32),
        pltpu.VMEM((NCHUNK, CH), jnp.float32),
        pltpu.VMEM((CH, H), jnp.float32),
        pltpu.VMEM((CH, H), jnp.float32),
        pltpu.VMEM((CH, H), jnp.float32),
        pltpu.SemaphoreType.DMA,
        pltpu.SemaphoreType.DMA,
        pltpu.SemaphoreType.DMA,
        pltpu.SemaphoreType.DMA,
        pltpu.SemaphoreType.DMA,
        pltpu.SemaphoreType.DMA,
    ],
    compiler_params=pltpu.CompilerParams(use_tc_tiling_on_sc=False),
)
def _agg_kernel(tab_hbm, src_hbm, dst_hbm, ew_hbm, out_hbm,
                acc_sh, srcv, dstv, ewv, r0, r1, r2,
                g0, g1, g2, s0, s1, s2):
    cid = lax.axis_index("c")
    sid = lax.axis_index("s")
    wid = sid * NC + cid
    rows = (r0, r1, r2)
    gsem = (g0, g1, g2)
    ssem = (s0, s1, s2)

    def zb(i, _):
        r0[i // (H // 16), pl.ds((i % (H // 16)) * 16, 16)] = (
            jnp.zeros((16,), jnp.float32))
        return 0
    lax.fori_loop(0, CH * (H // 16), zb, 0)

    def zc(j, _):
        pltpu.sync_copy(r0, acc_sh.at[pl.ds(sid * RPT + j * CH, CH)])
        return 0
    lax.fori_loop(0, RPT // CH, zc, 0)
    plsc.subcore_barrier()

    pltpu.sync_copy(src_hbm.at[pl.ds(wid * NCHUNK, NCHUNK)], srcv)
    pltpu.sync_copy(dst_hbm.at[pl.ds(wid * NCHUNK, NCHUNK)], dstv)
    pltpu.sync_copy(ew_hbm.at[pl.ds(wid * NCHUNK, NCHUNK)], ewv)

    pltpu.async_copy(tab_hbm.at[srcv.at[0]], r0, g0)
    pltpu.async_copy(tab_hbm.at[srcv.at[1]], r1, g1)

    def outer(h, _):
        for b in range(3):
            i = h * 3 + b
            bp = (b + 2) % 3
            rb, gb, sb = rows[b], gsem[b], ssem[b]
            pltpu.make_async_copy(tab_hbm.at[srcv.at[i]], rb, gb).wait()

            @pl.when(i >= 1)
            def _():
                pltpu.make_async_copy(
                    rows[bp], acc_sh.at[dstv.at[i - 1]], ssem[bp]).wait()

            @pl.when(i + 2 < NCHUNK)
            def _():
                pltpu.async_copy(tab_hbm.at[srcv.at[i + 2]], rows[bp],
                                 gsem[bp])

            def scale(g, _):
                ew16 = ewv[i, pl.ds(g * 16, 16)]
                for j in range(0, 16, 4):
                    e = g * 16 + j
                    vals = []
                    for k in range(4):
                        s = ew16[j + k]
                        vals.append([rb[e + k, pl.ds(f * 16, 16)] * s
                                     for f in range(H // 16)])
                    for k in range(4):
                        for f in range(H // 16):
                            rb[e + k, pl.ds(f * 16, 16)] = vals[k][f]
                return 0
            lax.fori_loop(0, CH // 16, scale, 0)

            pltpu.async_copy(rb, acc_sh.at[dstv.at[i]], sb, add=True)
        return 0
    lax.fori_loop(0, NCHUNK // 3, outer, 0)

    pltpu.make_async_copy(
        rows[(NCHUNK - 1) % 3], acc_sh.at[dstv.at[NCHUNK - 1]],
        ssem[(NCHUNK - 1) % 3]).wait()
    plsc.subcore_barrier()

    def dump(j, _):
        pltpu.sync_copy(acc_sh.at[pl.ds(sid * RPT + j * CH, CH)], r0)
        pltpu.sync_copy(r0, out_hbm.at[cid, pl.ds(sid * RPT + j * CH, CH)])
        return 0
    lax.fori_loop(0, RPT // CH, dump, 0)


BR = 2048
GRID = NP // BR


def _dinv(dga, dgb):
    deg = dga + dgb
    return jnp.where(deg > 0, lax.rsqrt(deg), 0.0)


def _dinv2(dg_ref, dg2_ref):
    deg = dg_ref[0] + dg2_ref[0]
    return jnp.where(deg > 0, lax.rsqrt(deg), 0.0)


def _mm1_body(x_ref, w_ref, dg_ref, dg2_ref, o_ref):
    dinv = _dinv2(dg_ref, dg2_ref)
    o_ref[...] = jnp.dot(x_ref[...], w_ref[...],
                         preferred_element_type=jnp.float32) * dinv


_mm1 = pl.pallas_call(
    _mm1_body,
    grid=(GRID,),
    in_specs=[
        pl.BlockSpec((BR, D), lambda i: (i, 0)),
        pl.BlockSpec((D, H), lambda i: (0, 0)),
        pl.BlockSpec((1, BR, 1), lambda i: (0, i, 0)),
        pl.BlockSpec((1, BR, 1), lambda i: (1, i, 0)),
    ],
    out_specs=pl.BlockSpec((BR, H), lambda i: (i, 0)),
    out_shape=jax.ShapeDtypeStruct((NP, H), jnp.float32),
)


def _mm2_body(a_ref, b_ref, dg_ref, dg2_ref, b1_ref, w_ref, o_ref):
    dinv = _dinv2(dg_ref, dg2_ref)
    h = jnp.maximum((a_ref[0] + b_ref[0]) * dinv + b1_ref[...], 0.0)
    o_ref[...] = jnp.dot(h, w_ref[...],
                         preferred_element_type=jnp.float32) * dinv


_mm2 = pl.pallas_call(
    _mm2_body,
    grid=(GRID,),
    in_specs=[
        pl.BlockSpec((1, BR, H), lambda i: (0, i, 0)),
        pl.BlockSpec((1, BR, H), lambda i: (1, i, 0)),
        pl.BlockSpec((1, BR, 1), lambda i: (0, i, 0)),
        pl.BlockSpec((1, BR, 1), lambda i: (1, i, 0)),
        pl.BlockSpec((1, H), lambda i: (0, 0)),
        pl.BlockSpec((H, H), lambda i: (0, 0)),
    ],
    out_specs=pl.BlockSpec((BR, H), lambda i: (i, 0)),
    out_shape=jax.ShapeDtypeStruct((NP, H), jnp.float32),
)


def _pool_body(a_ref, b_ref, dg_ref, dg2_ref, b2_ref, bat_ref, wl_ref,
               bl_ref, out_ref, pooled_ref, sums, cnts):
    i = pl.program_id(0)

    @pl.when(i == 0)
    def _():
        sums[...] = jnp.zeros_like(sums)
        cnts[...] = jnp.zeros_like(cnts)

    dinv = _dinv2(dg_ref, dg2_ref)
    h2 = jnp.maximum((a_ref[0] + b_ref[0]) * dinv + b2_ref[...], 0.0)
    sel = (bat_ref[...] == lax.broadcasted_iota(jnp.int32, (BR, G), 1)
           ).astype(jnp.float32)
    sums[...] += lax.dot_general(sel, h2, (((0,), (0,)), ((), ())),
                                 preferred_element_type=jnp.float32)
    cnts[...] += lax.dot_general(sel, jnp.ones((BR, 1), jnp.float32),
                                 (((0,), (0,)), ((), ())),
                                 preferred_element_type=jnp.float32)

    @pl.when(i == GRID - 1)
    def _():
        pooled = sums[...] / jnp.maximum(cnts[...], 1.0)
        pooled_ref[...] = pooled
        out_ref[...] = jnp.dot(pooled, wl_ref[...],
                               preferred_element_type=jnp.float32) + bl_ref[...]


_pool = pl.pallas_call(
    _pool_body,
    grid=(GRID,),
    in_specs=[
        pl.BlockSpec((1, BR, H), lambda i: (0, i, 0)),
        pl.BlockSpec((1, BR, H), lambda i: (1, i, 0)),
        pl.BlockSpec((1, BR, 1), lambda i: (0, i, 0)),
        pl.BlockSpec((1, BR, 1), lambda i: (1, i, 0)),
        pl.BlockSpec((1, H), lambda i: (0, 0)),
        pl.BlockSpec((BR, 1), lambda i: (i, 0)),
        pl.BlockSpec((H, C), lambda i: (0, 0)),
        pl.BlockSpec((1, C), lambda i: (0, 0)),
    ],
    out_specs=[
        pl.BlockSpec((G, C), lambda i: (0, 0)),
        pl.BlockSpec((G, H), lambda i: (0, 0)),
    ],
    out_shape=[
        jax.ShapeDtypeStruct((G, C), jnp.float32),
        jax.ShapeDtypeStruct((G, H), jnp.float32),
    ],
    scratch_shapes=[
        pltpu.VMEM((G, H), jnp.float32),
        pltpu.VMEM((G, 1), jnp.float32),
    ],
)


def kernel(x, edge_index, edge_weight, batch, W1, b1, W2, b2, Wl, bl):
    src, dst = edge_index[0], edge_index[1]
    loop_idx = jnp.arange(N, dtype=src.dtype)
    srcf = jnp.concatenate([src, loop_idx])
    dstf = jnp.concatenate([dst, loop_idx])
    ewf = jnp.concatenate([edge_weight, jnp.ones((N,), edge_weight.dtype)])

    pad = EP - EF
    srcf = jnp.pad(srcf, (0, pad)).reshape(EP // CH, CH)
    dstf = jnp.pad(dstf, (0, pad)).reshape(EP // CH, CH)
    ewf = jnp.pad(ewf, (0, pad)).reshape(EP // CH, CH)
    xp = jnp.pad(x, ((0, NP - N), (0, 0)))
    batp = jnp.pad(batch, (0, NP - N), constant_values=G).reshape(NP, 1)

    deg3 = _deg_kernel(dstf, ewf).reshape(NC, NP, 1)

    xs1 = _mm1(xp, W1, deg3, deg3)
    acc1 = _agg_kernel(xs1, srcf, dstf, ewf)
    xs2 = _mm2(acc1, acc1, deg3, deg3, b1.reshape(1, H), W2)
    acc2 = _agg_kernel(xs2, srcf, dstf, ewf)
    out, pooled = _pool(acc2, acc2, deg3, deg3, b2.reshape(1, H),
                        batp, Wl, bl.reshape(1, C))
    return (out, pooled)


# speedup vs baseline: 30.7004x; 1.0005x over previous
import functools

import jax
import jax.numpy as jnp
from jax import lax
from jax.experimental import pallas as pl
from jax.experimental.pallas import tpu as pltpu
from jax.experimental.pallas import tpu_sc as plsc

N = 10000
D = 128
H = 64
C = 2
G = 64
E = 320000

NC = 2
NS = 16
NW = NC * NS

NP = 10240
RPT = NP // NS
CH = 128
EF = E + N
_NC0 = (EF + NW * CH - 1) // (NW * CH)
NCHUNK = ((_NC0 + 2) // 3) * 3
ET = NCHUNK * CH
EP = NW * ET

_MESH = dict(core_axis_name="c", subcore_axis_name="s",
             num_cores=NC, num_subcores=NS)


@functools.partial(
    pl.kernel,
    out_type=jax.ShapeDtypeStruct((NC, NP), jnp.float32),
    mesh=plsc.VectorSubcoreMesh(**_MESH),
    scratch_types=[
        pltpu.VMEM_SHARED((NP,), jnp.float32),
        pltpu.VMEM((NCHUNK, CH), jnp.int32),
        pltpu.VMEM((NCHUNK, CH), jnp.float32),
        pltpu.VMEM((RPT,), jnp.float32),
        pltpu.SemaphoreType.DMA,
    ],
    compiler_params=pltpu.CompilerParams(use_tc_tiling_on_sc=False),
)
def _deg_kernel(dst_hbm, ew_hbm, out_hbm, deg_sh, dstv, ewv, buf, sem):
    cid = lax.axis_index("c")
    sid = lax.axis_index("s")
    wid = sid * NC + cid

    def zb(i, _):
        buf[pl.ds(i * 16, 16)] = jnp.zeros((16,), jnp.float32)
        return 0
    lax.fori_loop(0, RPT // 16, zb, 0)
    pltpu.sync_copy(buf, deg_sh.at[pl.ds(sid * RPT, RPT)])
    plsc.subcore_barrier()

    pltpu.sync_copy(dst_hbm.at[pl.ds(wid * NCHUNK, NCHUNK)], dstv)
    pltpu.sync_copy(ew_hbm.at[pl.ds(wid * NCHUNK, NCHUNK)], ewv)

    def fire(i, _):
        pltpu.sync_copy(ewv.at[i], deg_sh.at[dstv.at[i]], add=True)
        return 0
    lax.fori_loop(0, NCHUNK, fire, 0)
    plsc.subcore_barrier()

    pltpu.sync_copy(deg_sh.at[pl.ds(sid * RPT, RPT)], buf)
    pltpu.sync_copy(buf, out_hbm.at[cid, pl.ds(sid * RPT, RPT)])


@functools.partial(
    pl.kernel,
    out_type=jax.ShapeDtypeStruct((NC, NP, H), jnp.float32),
    mesh=plsc.VectorSubcoreMesh(**_MESH),
    scratch_types=[
        pltpu.VMEM_SHARED((NP, H), jnp.float32),
        pltpu.VMEM((NCHUNK, CH), jnp.int32),
        pltpu.VMEM((NCHUNK, CH), jnp.int32),
        pltpu.VMEM((NCHUNK, CH), jnp.float32),
        pltpu.VMEM((CH, H), jnp.float32),
        pltpu.VMEM((CH, H), jnp.float32),
        pltpu.VMEM((CH, H), jnp.float32),
        pltpu.SemaphoreType.DMA,
        pltpu.SemaphoreType.DMA,
        pltpu.SemaphoreType.DMA,
        pltpu.SemaphoreType.DMA,
        pltpu.SemaphoreType.DMA,
        pltpu.SemaphoreType.DMA,
    ],
    compiler_params=pltpu.CompilerParams(use_tc_tiling_on_sc=False),
)
def _agg_kernel(tab_hbm, src_hbm, dst_hbm, ew_hbm, out_hbm,
                acc_sh, srcv, dstv, ewv, r0, r1, r2,
                g0, g1, g2, s0, s1, s2):
    cid = lax.axis_index("c")
    sid = lax.axis_index("s")
    wid = sid * NC + cid
    rows = (r0, r1, r2)
    gsem = (g0, g1, g2)
    ssem = (s0, s1, s2)

    def zb(i, _):
        r0[i // (H // 16), pl.ds((i % (H // 16)) * 16, 16)] = (
            jnp.zeros((16,), jnp.float32))
        return 0
    lax.fori_loop(0, CH * (H // 16), zb, 0)

    def zc(j, _):
        pltpu.sync_copy(r0, acc_sh.at[pl.ds(sid * RPT + j * CH, CH)])
        return 0
    lax.fori_loop(0, RPT // CH, zc, 0)
    plsc.subcore_barrier()

    pltpu.sync_copy(src_hbm.at[pl.ds(wid * NCHUNK, NCHUNK)], srcv)
    pltpu.sync_copy(dst_hbm.at[pl.ds(wid * NCHUNK, NCHUNK)], dstv)
    pltpu.sync_copy(ew_hbm.at[pl.ds(wid * NCHUNK, NCHUNK)], ewv)

    pltpu.async_copy(tab_hbm.at[srcv.at[0]], r0, g0)
    pltpu.async_copy(tab_hbm.at[srcv.at[1]], r1, g1)

    def outer(h, _):
        for b in range(3):
            i = h * 3 + b
            bp = (b + 2) % 3
            rb, gb, sb = rows[b], gsem[b], ssem[b]
            pltpu.make_async_copy(tab_hbm.at[srcv.at[i]], rb, gb).wait()

            @pl.when(i >= 1)
            def _():
                pltpu.make_async_copy(
                    rows[bp], acc_sh.at[dstv.at[i - 1]], ssem[bp]).wait()

            @pl.when(i + 2 < NCHUNK)
            def _():
                pltpu.async_copy(tab_hbm.at[srcv.at[i + 2]], rows[bp],
                                 gsem[bp])

            def scale(g, _):
                ew16 = ewv[i, pl.ds(g * 16, 16)]
                for j in range(0, 16, 4):
                    e = g * 16 + j
                    vals = []
                    for k in range(4):
                        s = ew16[j + k]
                        vals.append([rb[e + k, pl.ds(f * 16, 16)] * s
                                     for f in range(H // 16)])
                    for k in range(4):
                        for f in range(H // 16):
                            rb[e + k, pl.ds(f * 16, 16)] = vals[k][f]
                return 0
            lax.fori_loop(0, CH // 16, scale, 0)

            pltpu.async_copy(rb, acc_sh.at[dstv.at[i]], sb, add=True)
        return 0
    lax.fori_loop(0, NCHUNK // 3, outer, 0)

    pltpu.make_async_copy(
        rows[(NCHUNK - 1) % 3], acc_sh.at[dstv.at[NCHUNK - 1]],
        ssem[(NCHUNK - 1) % 3]).wait()
    plsc.subcore_barrier()

    def dump(j, _):
        pltpu.sync_copy(acc_sh.at[pl.ds(sid * RPT + j * CH, CH)], r0)
        pltpu.sync_copy(r0, out_hbm.at[cid, pl.ds(sid * RPT + j * CH, CH)])
        return 0
    lax.fori_loop(0, RPT // CH, dump, 0)


BR = 2048
GRID = NP // BR


def _dinv(dga, dgb):
    deg = dga + dgb
    return jnp.where(deg > 0, lax.rsqrt(deg), 0.0)


def _dinv2(dg_ref):
    deg = dg_ref[0] + dg_ref[1]
    return jnp.where(deg > 0, lax.rsqrt(deg), 0.0)


def _mm1_body(x_ref, w_ref, dg_ref, o_ref):
    dinv = _dinv2(dg_ref)
    o_ref[...] = jnp.dot(x_ref[...], w_ref[...],
                         preferred_element_type=jnp.float32) * dinv


_mm1 = pl.pallas_call(
    _mm1_body,
    grid=(GRID,),
    in_specs=[
        pl.BlockSpec((BR, D), lambda i: (i, 0)),
        pl.BlockSpec((D, H), lambda i: (0, 0)),
        pl.BlockSpec((NC, BR, 1), lambda i: (0, i, 0)),
    ],
    out_specs=pl.BlockSpec((BR, H), lambda i: (i, 0)),
    out_shape=jax.ShapeDtypeStruct((NP, H), jnp.float32),
)


def _mm2_body(a_ref, dg_ref, b1_ref, w_ref, o_ref):
    dinv = _dinv2(dg_ref)
    h = jnp.maximum((a_ref[0] + a_ref[1]) * dinv + b1_ref[...], 0.0)
    o_ref[...] = jnp.dot(h, w_ref[...],
                         preferred_element_type=jnp.float32) * dinv


_mm2 = pl.pallas_call(
    _mm2_body,
    grid=(GRID,),
    in_specs=[
        pl.BlockSpec((NC, BR, H), lambda i: (0, i, 0)),
        pl.BlockSpec((NC, BR, 1), lambda i: (0, i, 0)),
        pl.BlockSpec((1, H), lambda i: (0, 0)),
        pl.BlockSpec((H, H), lambda i: (0, 0)),
    ],
    out_specs=pl.BlockSpec((BR, H), lambda i: (i, 0)),
    out_shape=jax.ShapeDtypeStruct((NP, H), jnp.float32),
)


def _pool_body(a_ref, dg_ref, b2_ref, bat_ref, wl_ref,
               bl_ref, out_ref, pooled_ref, sums, cnts):
    i = pl.program_id(0)

    @pl.when(i == 0)
    def _():
        sums[...] = jnp.zeros_like(sums)
        cnts[...] = jnp.zeros_like(cnts)

    dinv = _dinv2(dg_ref)
    h2 = jnp.maximum((a_ref[0] + a_ref[1]) * dinv + b2_ref[...], 0.0)
    sel = (bat_ref[...] == lax.broadcasted_iota(jnp.int32, (BR, G), 1)
           ).astype(jnp.float32)
    sums[...] += lax.dot_general(sel, h2, (((0,), (0,)), ((), ())),
                                 preferred_element_type=jnp.float32)
    cnts[...] += lax.dot_general(sel, jnp.ones((BR, 1), jnp.float32),
                                 (((0,), (0,)), ((), ())),
                                 preferred_element_type=jnp.float32)

    @pl.when(i == GRID - 1)
    def _():
        pooled = sums[...] / jnp.maximum(cnts[...], 1.0)
        pooled_ref[...] = pooled
        out_ref[...] = jnp.dot(pooled, wl_ref[...],
                               preferred_element_type=jnp.float32) + bl_ref[...]


_pool = pl.pallas_call(
    _pool_body,
    grid=(GRID,),
    in_specs=[
        pl.BlockSpec((NC, BR, H), lambda i: (0, i, 0)),
        pl.BlockSpec((NC, BR, 1), lambda i: (0, i, 0)),
        pl.BlockSpec((1, H), lambda i: (0, 0)),
        pl.BlockSpec((BR, 1), lambda i: (i, 0)),
        pl.BlockSpec((H, C), lambda i: (0, 0)),
        pl.BlockSpec((1, C), lambda i: (0, 0)),
    ],
    out_specs=[
        pl.BlockSpec((G, C), lambda i: (0, 0)),
        pl.BlockSpec((G, H), lambda i: (0, 0)),
    ],
    out_shape=[
        jax.ShapeDtypeStruct((G, C), jnp.float32),
        jax.ShapeDtypeStruct((G, H), jnp.float32),
    ],
    scratch_shapes=[
        pltpu.VMEM((G, H), jnp.float32),
        pltpu.VMEM((G, 1), jnp.float32),
    ],
)


def kernel(x, edge_index, edge_weight, batch, W1, b1, W2, b2, Wl, bl):
    src, dst = edge_index[0], edge_index[1]
    loop_idx = jnp.arange(N, dtype=src.dtype)
    srcf = jnp.concatenate([src, loop_idx])
    dstf = jnp.concatenate([dst, loop_idx])
    ewf = jnp.concatenate([edge_weight, jnp.ones((N,), edge_weight.dtype)])

    pad = EP - EF
    srcf = jnp.pad(srcf, (0, pad)).reshape(EP // CH, CH)
    dstf = jnp.pad(dstf, (0, pad)).reshape(EP // CH, CH)
    ewf = jnp.pad(ewf, (0, pad)).reshape(EP // CH, CH)
    xp = jnp.pad(x, ((0, NP - N), (0, 0)))
    batp = jnp.pad(batch, (0, NP - N), constant_values=G).reshape(NP, 1)

    deg3 = _deg_kernel(dstf, ewf).reshape(NC, NP, 1)

    xs1 = _mm1(xp, W1, deg3)
    acc1 = _agg_kernel(xs1, srcf, dstf, ewf)
    xs2 = _mm2(acc1, deg3, b1.reshape(1, H), W2)
    acc2 = _agg_kernel(xs2, srcf, dstf, ewf)
    out, pooled = _pool(acc2, deg3, b2.reshape(1, H),
                        batp, Wl, bl.reshape(1, C))
    return (out, pooled)
